# Initial kernel scaffold; baseline (speedup 1.0000x reference)
import jax
import jax.numpy as jnp
from jax.experimental import pallas as pl


def kernel(x, edge_index, W_feat, b_feat, group_encodings, W_agg1, b_agg1, W_agg2, b_agg2, W_c1, b_c1, W_c2, b_c2):
    raise NotImplementedError("write your pallas kernel here")


# speedup vs baseline: 3.5370x; 3.5370x over previous
import jax
import jax.numpy as jnp
from jax import lax
from jax.experimental import pallas as pl
from jax.experimental.pallas import tpu as pltpu
from jax.experimental.pallas import tpu_sc as plsc

N = 10000
D = 128
NC = 2
NS = 16
NW = NC * NS
CHUNK = 128
CH = 80
E_PAD = NW * CH * CHUNK
N_ACC = 10240
ZR = N_ACC // NS
ROWBLK = 1280


def _lin_relu_body(x_ref, w_ref, b_ref, o_ref):
    o_ref[...] = jnp.maximum(
        jnp.dot(x_ref[...], w_ref[...], preferred_element_type=jnp.float32)
        + b_ref[...], 0.0)


def _tc_linear_relu(x, w, b):
    n = x.shape[0]
    blk = 1000
    return pl.pallas_call(
        _lin_relu_body,
        grid=(n // blk,),
        in_specs=[pl.BlockSpec((blk, D), lambda i: (i, 0)),
                  pl.BlockSpec((D, D), lambda i: (0, 0)),
                  pl.BlockSpec((1, D), lambda i: (0, 0))],
        out_specs=pl.BlockSpec((blk, D), lambda i: (i, 0)),
        out_shape=jax.ShapeDtypeStruct((n, D), jnp.float32),
    )(x, w, b.reshape(1, D))


def _norm_lin_relu_body(s_ref, c_ref, w_ref, b_ref, o_ref):
    cnt = (c_ref[0] + c_ref[1]).reshape(-1, 1)
    m = (s_ref[0] + s_ref[1]) / jnp.maximum(cnt, 1.0)
    o_ref[...] = jnp.maximum(
        jnp.dot(m, w_ref[...], preferred_element_type=jnp.float32)
        + b_ref[...], 0.0)


def _tc_norm_linear_relu(sums, cnt, w, b):
    return pl.pallas_call(
        _norm_lin_relu_body,
        grid=(N_ACC // ROWBLK,),
        in_specs=[pl.BlockSpec((NC, ROWBLK, D), lambda i: (0, i, 0)),
                  pl.BlockSpec((NC, ROWBLK), lambda i: (0, i)),
                  pl.BlockSpec((D, D), lambda i: (0, 0)),
                  pl.BlockSpec((1, D), lambda i: (0, 0))],
        out_specs=pl.BlockSpec((ROWBLK, D), lambda i: (i, 0)),
        out_shape=jax.ShapeDtypeStruct((N_ACC, D), jnp.float32),
    )(sums, cnt, w, b.reshape(1, D))


def _final_body(s_ref, c_ref, w2_ref, b2_ref, wc1_ref, bc1_ref, wc2_ref,
                bc2_ref, o_ref):
    cnt = (c_ref[0] + c_ref[1]).reshape(-1, 1)
    m = (s_ref[0] + s_ref[1]) / jnp.maximum(cnt, 1.0)
    h2 = jnp.dot(m, w2_ref[...], preferred_element_type=jnp.float32) + b2_ref[...]
    hid = jnp.maximum(
        jnp.dot(h2, wc1_ref[...], preferred_element_type=jnp.float32)
        + bc1_ref[...], 0.0)
    o_ref[...] = jnp.dot(hid, wc2_ref[...],
                         preferred_element_type=jnp.float32) + bc2_ref[...]


def _tc_final(sums, cnt, w2, b2, wc1, bc1, wc2, bc2):
    h2w = wc1.shape[0]
    hid = wc1.shape[1]
    out = wc2.shape[1]
    return pl.pallas_call(
        _final_body,
        grid=(N_ACC // ROWBLK,),
        in_specs=[pl.BlockSpec((NC, ROWBLK, D), lambda i: (0, i, 0)),
                  pl.BlockSpec((NC, ROWBLK), lambda i: (0, i)),
                  pl.BlockSpec((D, D), lambda i: (0, 0)),
                  pl.BlockSpec((1, D), lambda i: (0, 0)),
                  pl.BlockSpec((h2w, hid), lambda i: (0, 0)),
                  pl.BlockSpec((1, hid), lambda i: (0, 0)),
                  pl.BlockSpec((hid, out), lambda i: (0, 0)),
                  pl.BlockSpec((1, out), lambda i: (0, 0))],
        out_specs=pl.BlockSpec((ROWBLK, out), lambda i: (i, 0)),
        out_shape=jax.ShapeDtypeStruct((N_ACC, out), jnp.float32),
    )(sums, cnt, w2, b2.reshape(1, D), wc1, bc1.reshape(1, hid),
      wc2, bc2.reshape(1, out))


_MESH = plsc.VectorSubcoreMesh(core_axis_name="c", subcore_axis_name="s",
                               num_cores=NC, num_subcores=NS)


def _sc_agg_cnt_body(h_hbm, src_hbm, dst_hbm, z2_hbm, z1_hbm,
                     sums_out, cnt_out, acc, cacc, idxs, idxd, rows, ones,
                     gsem):
    cid = lax.axis_index("c")
    sid = lax.axis_index("s")
    wid = sid * NC + cid
    zbase = sid * ZR
    pltpu.sync_copy(z2_hbm, acc.at[pl.ds(zbase, ZR)])
    pltpu.sync_copy(z1_hbm, cacc.at[pl.ds(zbase, ZR)])
    for i in range(CHUNK // 16):
        ones[pl.ds(i * 16, 16)] = jnp.ones((16,), jnp.float32)
    pltpu.sync_copy(src_hbm.at[wid], idxs)
    pltpu.sync_copy(dst_hbm.at[wid], idxd)
    plsc.subcore_barrier()

    def chunk(j, carry):
        pltpu.async_copy(h_hbm.at[idxs.at[j]], rows, gsem).wait()
        pltpu.sync_copy(rows, acc.at[idxd.at[j]], add=True)
        pltpu.sync_copy(ones, cacc.at[idxd.at[j]], add=True)
        return carry

    lax.fori_loop(0, CH, chunk, 0)
    plsc.subcore_barrier()
    pltpu.sync_copy(acc.at[pl.ds(zbase, ZR)],
                    sums_out.at[cid, pl.ds(zbase, ZR)])
    pltpu.sync_copy(cacc.at[pl.ds(zbase, ZR)],
                    cnt_out.at[cid, pl.ds(zbase, ZR)])


def _sc_agg_body(h_hbm, src_hbm, dst_hbm, z2_hbm,
                 sums_out, acc, idxs, idxd, rows, gsem):
    cid = lax.axis_index("c")
    sid = lax.axis_index("s")
    wid = sid * NC + cid
    zbase = sid * ZR
    pltpu.sync_copy(z2_hbm, acc.at[pl.ds(zbase, ZR)])
    pltpu.sync_copy(src_hbm.at[wid], idxs)
    pltpu.sync_copy(dst_hbm.at[wid], idxd)
    plsc.subcore_barrier()

    def chunk(j, carry):
        pltpu.async_copy(h_hbm.at[idxs.at[j]], rows, gsem).wait()
        pltpu.sync_copy(rows, acc.at[idxd.at[j]], add=True)
        return carry

    lax.fori_loop(0, CH, chunk, 0)
    plsc.subcore_barrier()
    pltpu.sync_copy(acc.at[pl.ds(zbase, ZR)],
                    sums_out.at[cid, pl.ds(zbase, ZR)])


_SC_AGG_CNT = pl.kernel(
    _sc_agg_cnt_body,
    out_type=(jax.ShapeDtypeStruct((NC, N_ACC, D), jnp.float32),
              jax.ShapeDtypeStruct((NC, N_ACC), jnp.float32)),
    mesh=_MESH,
    scratch_types=[
        pltpu.VMEM_SHARED((N_ACC, D), jnp.float32),
        pltpu.VMEM_SHARED((N_ACC,), jnp.float32),
        pltpu.VMEM((CH, CHUNK), jnp.int32),
        pltpu.VMEM((CH, CHUNK), jnp.int32),
        pltpu.VMEM((CHUNK, D), jnp.float32),
        pltpu.VMEM((CHUNK,), jnp.float32),
        pltpu.SemaphoreType.DMA,
    ],
)

_SC_AGG = pl.kernel(
    _sc_agg_body,
    out_type=jax.ShapeDtypeStruct((NC, N_ACC, D), jnp.float32),
    mesh=_MESH,
    scratch_types=[
        pltpu.VMEM_SHARED((N_ACC, D), jnp.float32),
        pltpu.VMEM((CH, CHUNK), jnp.int32),
        pltpu.VMEM((CH, CHUNK), jnp.int32),
        pltpu.VMEM((CHUNK, D), jnp.float32),
        pltpu.SemaphoreType.DMA,
    ],
)


def kernel(x, edge_index, W_feat, b_feat, group_encodings, W_agg1, b_agg1,
           W_agg2, b_agg2, W_c1, b_c1, W_c2, b_c2):
    src = edge_index[0]
    dst = edge_index[1]
    e = src.shape[0]
    pad = E_PAD - e
    src3d = jnp.concatenate(
        [src, jnp.zeros((pad,), jnp.int32)]).reshape(NW, CH, CHUNK)
    dst3d = jnp.concatenate(
        [dst, jnp.full((pad,), N, jnp.int32)]).reshape(NW, CH, CHUNK)
    z2 = jnp.zeros((ZR, D), jnp.float32)
    z1 = jnp.zeros((ZR,), jnp.float32)
    b_eff = b_agg2 + jnp.mean(group_encodings, axis=0)

    h = _tc_linear_relu(x, W_feat, b_feat)
    sums1, cnt = _SC_AGG_CNT(h, src3d, dst3d, z2, z1)
    h1 = _tc_norm_linear_relu(sums1, cnt, W_agg1, b_agg1)
    sums2 = _SC_AGG(h1, src3d, dst3d, z2)
    out = _tc_final(sums2, cnt, W_agg2, b_eff, W_c1, b_c1, W_c2, b_c2)
    return out[:N]


# speedup vs baseline: 3.6934x; 1.0442x over previous
import jax
import jax.numpy as jnp
from jax import lax
from jax.experimental import pallas as pl
from jax.experimental.pallas import tpu as pltpu
from jax.experimental.pallas import tpu_sc as plsc

N = 10000
D = 128
NC = 2
NS = 16
NW = NC * NS
CHUNK = 128
CH = 80
E_PAD = NW * CH * CHUNK
N_ACC = 10240
ZR = N_ACC // NS
ROWBLK = 1280


def _lin_relu_body(x_ref, w_ref, b_ref, o_ref):
    o_ref[...] = jnp.maximum(
        jnp.dot(x_ref[...], w_ref[...], preferred_element_type=jnp.float32)
        + b_ref[...], 0.0)


def _tc_linear_relu(x, w, b):
    n = x.shape[0]
    blk = 1000
    return pl.pallas_call(
        _lin_relu_body,
        grid=(n // blk,),
        in_specs=[pl.BlockSpec((blk, D), lambda i: (i, 0)),
                  pl.BlockSpec((D, D), lambda i: (0, 0)),
                  pl.BlockSpec((1, D), lambda i: (0, 0))],
        out_specs=pl.BlockSpec((blk, D), lambda i: (i, 0)),
        out_shape=jax.ShapeDtypeStruct((n, D), jnp.float32),
    )(x, w, b.reshape(1, D))


def _norm_lin_relu_body(s_ref, c_ref, w_ref, b_ref, o_ref):
    cnt = c_ref[0] + c_ref[1]
    m = (s_ref[0] + s_ref[1]) / jnp.maximum(cnt, 1.0)
    o_ref[...] = jnp.maximum(
        jnp.dot(m, w_ref[...], preferred_element_type=jnp.float32)
        + b_ref[...], 0.0)


def _tc_norm_linear_relu(sums, cnt, w, b):
    return pl.pallas_call(
        _norm_lin_relu_body,
        grid=(N_ACC // ROWBLK,),
        in_specs=[pl.BlockSpec((NC, ROWBLK, D), lambda i: (0, i, 0)),
                  pl.BlockSpec((NC, ROWBLK, 1), lambda i: (0, i, 0)),
                  pl.BlockSpec((D, D), lambda i: (0, 0)),
                  pl.BlockSpec((1, D), lambda i: (0, 0))],
        out_specs=pl.BlockSpec((ROWBLK, D), lambda i: (i, 0)),
        out_shape=jax.ShapeDtypeStruct((N_ACC, D), jnp.float32),
    )(sums, cnt.reshape(NC, N_ACC, 1), w, b.reshape(1, D))


def _final_body(s_ref, c_ref, w2_ref, b2_ref, wc1_ref, bc1_ref, wc2_ref,
                bc2_ref, o_ref):
    cnt = c_ref[0] + c_ref[1]
    m = (s_ref[0] + s_ref[1]) / jnp.maximum(cnt, 1.0)
    h2 = jnp.dot(m, w2_ref[...], preferred_element_type=jnp.float32) + b2_ref[...]
    hid = jnp.maximum(
        jnp.dot(h2, wc1_ref[...], preferred_element_type=jnp.float32)
        + bc1_ref[...], 0.0)
    o_ref[...] = jnp.dot(hid, wc2_ref[...],
                         preferred_element_type=jnp.float32) + bc2_ref[...]


def _tc_final(sums, cnt, w2, b2, wc1, bc1, wc2, bc2):
    h2w = wc1.shape[0]
    hid = wc1.shape[1]
    out = wc2.shape[1]
    return pl.pallas_call(
        _final_body,
        grid=(N_ACC // ROWBLK,),
        in_specs=[pl.BlockSpec((NC, ROWBLK, D), lambda i: (0, i, 0)),
                  pl.BlockSpec((NC, ROWBLK, 1), lambda i: (0, i, 0)),
                  pl.BlockSpec((D, D), lambda i: (0, 0)),
                  pl.BlockSpec((1, D), lambda i: (0, 0)),
                  pl.BlockSpec((h2w, hid), lambda i: (0, 0)),
                  pl.BlockSpec((1, hid), lambda i: (0, 0)),
                  pl.BlockSpec((hid, out), lambda i: (0, 0)),
                  pl.BlockSpec((1, out), lambda i: (0, 0))],
        out_specs=pl.BlockSpec((ROWBLK, out), lambda i: (i, 0)),
        out_shape=jax.ShapeDtypeStruct((N_ACC, out), jnp.float32),
    )(sums, cnt.reshape(NC, N_ACC, 1), w2, b2.reshape(1, D),
      wc1, bc1.reshape(1, hid), wc2, bc2.reshape(1, out))


_MESH = plsc.VectorSubcoreMesh(core_axis_name="c", subcore_axis_name="s",
                               num_cores=NC, num_subcores=NS)


NBUF = 2
IH = 40
G_STEPS = IH // NBUF


def _agg_pipeline(h_hbm, acc, cacc, idxs, idxd, rows, ones, gsems, ssems,
                  csems, with_count):
    def start_gather(j, b):
        pltpu.async_copy(h_hbm.at[idxs.at[j]], rows.at[b], gsems.at[b])

    def wait_gather(b):
        pltpu.make_async_copy(h_hbm.at[idxs.at[0]], rows.at[b],
                              gsems.at[b]).wait()

    def start_scatter(j, b):
        pltpu.async_copy(rows.at[b], acc.at[idxd.at[j]], ssems.at[b],
                         add=True)
        if with_count:
            pltpu.async_copy(ones, cacc.at[idxd.at[j]], csems.at[b],
                             add=True)

    def wait_scatter(b):
        pltpu.make_async_copy(rows.at[b], acc.at[idxd.at[0]],
                              ssems.at[b]).wait()
        if with_count:
            pltpu.make_async_copy(ones, cacc.at[idxd.at[0]],
                                  csems.at[b]).wait()

    for b in range(NBUF):
        start_gather(b, b)

    def step(g, carry):
        for b in range(NBUF):
            wait_gather(b)
            start_scatter(g * NBUF + b, b)
        for b in range(NBUF):
            wait_scatter(b)
            start_gather((g + 1) * NBUF + b, b)
        return carry

    lax.fori_loop(0, G_STEPS - 1, step, 0)
    g = G_STEPS - 1
    for b in range(NBUF):
        wait_gather(b)
        start_scatter(g * NBUF + b, b)
    for b in range(NBUF):
        wait_scatter(b)


def _sc_agg_cnt_body(h_hbm, src_hbm, dst_hbm, z2_hbm, z1_hbm,
                     sums_out, cnt_out0, cnt_out1, acc, cacc, idxs, idxd,
                     rows, ones, gsems, ssems, csems):
    cid = lax.axis_index("c")
    sid = lax.axis_index("s")
    wid = sid * NC + cid
    zbase = sid * ZR
    pltpu.sync_copy(z2_hbm, acc.at[pl.ds(zbase, ZR)])
    pltpu.sync_copy(z1_hbm, cacc.at[pl.ds(zbase, ZR)])
    for i in range(CHUNK // 16):
        ones[pl.ds(i * 16, 16)] = jnp.ones((16,), jnp.float32)
    plsc.subcore_barrier()
    for half in range(CH // IH):
        pltpu.sync_copy(src_hbm.at[wid, pl.ds(half * IH, IH)], idxs)
        pltpu.sync_copy(dst_hbm.at[wid, pl.ds(half * IH, IH)], idxd)
        _agg_pipeline(h_hbm, acc, cacc, idxs, idxd, rows, ones, gsems,
                      ssems, csems, with_count=True)
    plsc.subcore_barrier()
    pltpu.sync_copy(acc.at[pl.ds(zbase, ZR)],
                    sums_out.at[cid, pl.ds(zbase, ZR)])

    @pl.when(cid == 0)
    def _():
        pltpu.sync_copy(cacc.at[pl.ds(zbase, ZR)],
                        cnt_out0.at[pl.ds(zbase, ZR)])

    @pl.when(cid == 1)
    def _():
        pltpu.sync_copy(cacc.at[pl.ds(zbase, ZR)],
                        cnt_out1.at[pl.ds(zbase, ZR)])


def _sc_agg_body(h_hbm, src_hbm, dst_hbm, z2_hbm,
                 sums_out, acc, idxs, idxd, rows, gsems, ssems):
    cid = lax.axis_index("c")
    sid = lax.axis_index("s")
    wid = sid * NC + cid
    zbase = sid * ZR
    pltpu.sync_copy(z2_hbm, acc.at[pl.ds(zbase, ZR)])
    plsc.subcore_barrier()
    for half in range(CH // IH):
        pltpu.sync_copy(src_hbm.at[wid, pl.ds(half * IH, IH)], idxs)
        pltpu.sync_copy(dst_hbm.at[wid, pl.ds(half * IH, IH)], idxd)
        _agg_pipeline(h_hbm, acc, None, idxs, idxd, rows, None, gsems,
                      ssems, None, with_count=False)
    plsc.subcore_barrier()
    pltpu.sync_copy(acc.at[pl.ds(zbase, ZR)],
                    sums_out.at[cid, pl.ds(zbase, ZR)])


_SC_AGG_CNT = pl.kernel(
    _sc_agg_cnt_body,
    out_type=(jax.ShapeDtypeStruct((NC, N_ACC, D), jnp.float32),
              jax.ShapeDtypeStruct((N_ACC,), jnp.float32),
              jax.ShapeDtypeStruct((N_ACC,), jnp.float32)),
    mesh=_MESH,
    scratch_types=[
        pltpu.VMEM_SHARED((N_ACC, D), jnp.float32),
        pltpu.VMEM_SHARED((N_ACC,), jnp.float32),
        pltpu.VMEM((IH, CHUNK), jnp.int32),
        pltpu.VMEM((IH, CHUNK), jnp.int32),
        pltpu.VMEM((NBUF, CHUNK, D), jnp.float32),
        pltpu.VMEM((CHUNK,), jnp.float32),
        pltpu.SemaphoreType.DMA((NBUF,)),
        pltpu.SemaphoreType.DMA((NBUF,)),
        pltpu.SemaphoreType.DMA((NBUF,)),
    ],
)

_SC_AGG = pl.kernel(
    _sc_agg_body,
    out_type=jax.ShapeDtypeStruct((NC, N_ACC, D), jnp.float32),
    mesh=_MESH,
    scratch_types=[
        pltpu.VMEM_SHARED((N_ACC, D), jnp.float32),
        pltpu.VMEM((IH, CHUNK), jnp.int32),
        pltpu.VMEM((IH, CHUNK), jnp.int32),
        pltpu.VMEM((NBUF, CHUNK, D), jnp.float32),
        pltpu.SemaphoreType.DMA((NBUF,)),
        pltpu.SemaphoreType.DMA((NBUF,)),
    ],
)


def kernel(x, edge_index, W_feat, b_feat, group_encodings, W_agg1, b_agg1,
           W_agg2, b_agg2, W_c1, b_c1, W_c2, b_c2):
    src = edge_index[0]
    dst = edge_index[1]
    e = src.shape[0]
    pad = E_PAD - e
    src3d = jnp.concatenate(
        [src, jnp.zeros((pad,), jnp.int32)]).reshape(NW, CH, CHUNK)
    dst3d = jnp.concatenate(
        [dst, jnp.full((pad,), N, jnp.int32)]).reshape(NW, CH, CHUNK)
    z2 = jnp.zeros((ZR, D), jnp.float32)
    z1 = jnp.zeros((ZR,), jnp.float32)
    b_eff = b_agg2 + jnp.mean(group_encodings, axis=0)

    h = _tc_linear_relu(x, W_feat, b_feat)
    sums1, cnt0, cnt1 = _SC_AGG_CNT(h, src3d, dst3d, z2, z1)
    cnt = jnp.stack([cnt0, cnt1], axis=0)
    h1 = _tc_norm_linear_relu(sums1, cnt, W_agg1, b_agg1)
    sums2 = _SC_AGG(h1, src3d, dst3d, z2)
    out = _tc_final(sums2, cnt, W_agg2, b_eff, W_c1, b_c1, W_c2, b_c2)
    return out[:N]


# speedup vs baseline: 10.5569x; 2.8583x over previous
import jax
import jax.numpy as jnp
from jax import lax
from jax.experimental import pallas as pl
from jax.experimental.pallas import tpu as pltpu
from jax.experimental.pallas import tpu_sc as plsc

N = 10000
D = 128
NC = 2
NS = 16
NW = NC * NS
CHUNK = 128
CH = 80
E_PAD = NW * CH * CHUNK
N_ACC = 10240
ZR = N_ACC // NS
ROWBLK = 1280


def _lin_relu_body(x_ref, w_ref, b_ref, o_ref):
    o_ref[...] = jnp.maximum(
        jnp.dot(x_ref[...], w_ref[...], preferred_element_type=jnp.float32)
        + b_ref[...], 0.0)


def _tc_linear_relu(x, w, b):
    n = x.shape[0]
    blk = 1000
    return pl.pallas_call(
        _lin_relu_body,
        grid=(n // blk,),
        in_specs=[pl.BlockSpec((blk, D), lambda i: (i, 0)),
                  pl.BlockSpec((D, D), lambda i: (0, 0)),
                  pl.BlockSpec((1, D), lambda i: (0, 0))],
        out_specs=pl.BlockSpec((blk, D), lambda i: (i, 0)),
        out_shape=jax.ShapeDtypeStruct((n, D), jnp.float32),
    )(x, w, b.reshape(1, D))


def _norm_lin_relu_body(s_ref, c_ref, w_ref, b_ref, o_ref):
    cnt = c_ref[0] + c_ref[1]
    m = (s_ref[0] + s_ref[1]) / jnp.maximum(cnt, 1.0)
    o_ref[...] = jnp.maximum(
        jnp.dot(m, w_ref[...], preferred_element_type=jnp.float32)
        + b_ref[...], 0.0)


def _tc_norm_linear_relu(sums, cnt, w, b):
    return pl.pallas_call(
        _norm_lin_relu_body,
        grid=(N_ACC // ROWBLK,),
        in_specs=[pl.BlockSpec((NC, ROWBLK, D), lambda i: (0, i, 0)),
                  pl.BlockSpec((NC, ROWBLK, 1), lambda i: (0, i, 0)),
                  pl.BlockSpec((D, D), lambda i: (0, 0)),
                  pl.BlockSpec((1, D), lambda i: (0, 0))],
        out_specs=pl.BlockSpec((ROWBLK, D), lambda i: (i, 0)),
        out_shape=jax.ShapeDtypeStruct((N_ACC, D), jnp.float32),
    )(sums, cnt.reshape(NC, N_ACC, 1), w, b.reshape(1, D))


def _final_body(s_ref, c_ref, w2_ref, b2_ref, wc1_ref, bc1_ref, wc2_ref,
                bc2_ref, o_ref):
    cnt = c_ref[0] + c_ref[1]
    m = (s_ref[0] + s_ref[1]) / jnp.maximum(cnt, 1.0)
    h2 = jnp.dot(m, w2_ref[...], preferred_element_type=jnp.float32) + b2_ref[...]
    hid = jnp.maximum(
        jnp.dot(h2, wc1_ref[...], preferred_element_type=jnp.float32)
        + bc1_ref[...], 0.0)
    o_ref[...] = jnp.dot(hid, wc2_ref[...],
                         preferred_element_type=jnp.float32) + bc2_ref[...]


def _tc_final(sums, cnt, w2, b2, wc1, bc1, wc2, bc2):
    h2w = wc1.shape[0]
    hid = wc1.shape[1]
    out = wc2.shape[1]
    return pl.pallas_call(
        _final_body,
        grid=(N_ACC // ROWBLK,),
        in_specs=[pl.BlockSpec((NC, ROWBLK, D), lambda i: (0, i, 0)),
                  pl.BlockSpec((NC, ROWBLK, 1), lambda i: (0, i, 0)),
                  pl.BlockSpec((D, D), lambda i: (0, 0)),
                  pl.BlockSpec((1, D), lambda i: (0, 0)),
                  pl.BlockSpec((h2w, hid), lambda i: (0, 0)),
                  pl.BlockSpec((1, hid), lambda i: (0, 0)),
                  pl.BlockSpec((hid, out), lambda i: (0, 0)),
                  pl.BlockSpec((1, out), lambda i: (0, 0))],
        out_specs=pl.BlockSpec((ROWBLK, out), lambda i: (i, 0)),
        out_shape=jax.ShapeDtypeStruct((N_ACC, out), jnp.float32),
    )(sums, cnt.reshape(NC, N_ACC, 1), w2, b2.reshape(1, D),
      wc1, bc1.reshape(1, hid), wc2, bc2.reshape(1, out))


_MESH = plsc.VectorSubcoreMesh(core_axis_name="c", subcore_axis_name="s",
                               num_cores=NC, num_subcores=NS)


NBUF = 2
IH = 40
G_STEPS = IH // NBUF


def _agg_pipeline(h_hbm, acc, cacc, idxs, idxd, rows, ones, gsems, ssems,
                  csems, with_count):
    def start_gather(j, b):
        pltpu.async_copy(h_hbm.at[idxs.at[j]], rows.at[b], gsems.at[b])

    def wait_gather(b):
        pltpu.make_async_copy(h_hbm.at[idxs.at[0]], rows.at[b],
                              gsems.at[b]).wait()

    def start_scatter(j, b):
        pltpu.async_copy(rows.at[b], acc.at[idxd.at[j]], ssems.at[b],
                         add=True)
        if with_count:
            pltpu.async_copy(ones, cacc.at[idxd.at[j]], csems.at[b],
                             add=True)

    def wait_scatter(b):
        pltpu.make_async_copy(rows.at[b], acc.at[idxd.at[0]],
                              ssems.at[b]).wait()
        if with_count:
            pltpu.make_async_copy(ones, cacc.at[idxd.at[0]],
                                  csems.at[b]).wait()

    for b in range(NBUF):
        start_gather(b, b)

    def step(g, carry):
        for b in range(NBUF):
            wait_gather(b)
            start_scatter(g * NBUF + b, b)
        for b in range(NBUF):
            wait_scatter(b)
            start_gather((g + 1) * NBUF + b, b)
        return carry

    lax.fori_loop(0, G_STEPS - 1, step, 0)
    g = G_STEPS - 1
    for b in range(NBUF):
        wait_gather(b)
        start_scatter(g * NBUF + b, b)
    for b in range(NBUF):
        wait_scatter(b)


def _sc_agg_cnt_body(h_hbm, src_hbm, dst_hbm, z2_hbm, z1_hbm,
                     sums_out, cnt_out0, cnt_out1, acc, cacc, idxs, idxd,
                     rows, ones, gsems, ssems, csems):
    cid = lax.axis_index("c")
    sid = lax.axis_index("s")
    wid = sid * NC + cid
    zbase = sid * ZR
    pltpu.sync_copy(z2_hbm, acc.at[pl.ds(zbase, ZR)])
    pltpu.sync_copy(z1_hbm, cacc.at[pl.ds(zbase, ZR)])
    for i in range(CHUNK // 16):
        ones[pl.ds(i * 16, 16)] = jnp.ones((16,), jnp.float32)
    plsc.subcore_barrier()
    for half in range(CH // IH):
        pltpu.sync_copy(src_hbm.at[wid, pl.ds(half * IH, IH)], idxs)
        pltpu.sync_copy(dst_hbm.at[wid, pl.ds(half * IH, IH)], idxd)
        _agg_pipeline(h_hbm, acc, cacc, idxs, idxd, rows, ones, gsems,
                      ssems, csems, with_count=True)
    plsc.subcore_barrier()
    pltpu.sync_copy(acc.at[pl.ds(zbase, ZR)],
                    sums_out.at[cid, pl.ds(zbase, ZR)])

    @pl.when(cid == 0)
    def _():
        pltpu.sync_copy(cacc.at[pl.ds(zbase, ZR)],
                        cnt_out0.at[pl.ds(zbase, ZR)])

    @pl.when(cid == 1)
    def _():
        pltpu.sync_copy(cacc.at[pl.ds(zbase, ZR)],
                        cnt_out1.at[pl.ds(zbase, ZR)])


def _sc_agg_body(h_hbm, src_hbm, dst_hbm, z2_hbm,
                 sums_out, acc, idxs, idxd, rows, gsems, ssems):
    cid = lax.axis_index("c")
    sid = lax.axis_index("s")
    wid = sid * NC + cid
    zbase = sid * ZR
    pltpu.sync_copy(z2_hbm, acc.at[pl.ds(zbase, ZR)])
    plsc.subcore_barrier()
    for half in range(CH // IH):
        pltpu.sync_copy(src_hbm.at[wid, pl.ds(half * IH, IH)], idxs)
        pltpu.sync_copy(dst_hbm.at[wid, pl.ds(half * IH, IH)], idxd)
        _agg_pipeline(h_hbm, acc, None, idxs, idxd, rows, None, gsems,
                      ssems, None, with_count=False)
    plsc.subcore_barrier()
    pltpu.sync_copy(acc.at[pl.ds(zbase, ZR)],
                    sums_out.at[cid, pl.ds(zbase, ZR)])


_SC_AGG_CNT = pl.kernel(
    _sc_agg_cnt_body,
    out_type=(jax.ShapeDtypeStruct((NC, N_ACC, D), jnp.float32),
              jax.ShapeDtypeStruct((N_ACC,), jnp.float32),
              jax.ShapeDtypeStruct((N_ACC,), jnp.float32)),
    mesh=_MESH,
    scratch_types=[
        pltpu.VMEM_SHARED((N_ACC, D), jnp.float32),
        pltpu.VMEM_SHARED((N_ACC,), jnp.float32),
        pltpu.VMEM((IH, CHUNK), jnp.int32),
        pltpu.VMEM((IH, CHUNK), jnp.int32),
        pltpu.VMEM((NBUF, CHUNK, D), jnp.float32),
        pltpu.VMEM((CHUNK,), jnp.float32),
        pltpu.SemaphoreType.DMA((NBUF,)),
        pltpu.SemaphoreType.DMA((NBUF,)),
        pltpu.SemaphoreType.DMA((NBUF,)),
    ],
)

_SC_AGG = pl.kernel(
    _sc_agg_body,
    out_type=jax.ShapeDtypeStruct((NC, N_ACC, D), jnp.float32),
    mesh=_MESH,
    scratch_types=[
        pltpu.VMEM_SHARED((N_ACC, D), jnp.float32),
        pltpu.VMEM((IH, CHUNK), jnp.int32),
        pltpu.VMEM((IH, CHUNK), jnp.int32),
        pltpu.VMEM((NBUF, CHUNK, D), jnp.float32),
        pltpu.SemaphoreType.DMA((NBUF,)),
        pltpu.SemaphoreType.DMA((NBUF,)),
    ],
)


def kernel(x, edge_index, W_feat, b_feat, group_encodings, W_agg1, b_agg1,
           W_agg2, b_agg2, W_c1, b_c1, W_c2, b_c2):
    src = edge_index[0]
    dst = edge_index[1]
    e = src.shape[0]
    pad = E_PAD - e
    ar = lax.iota(jnp.int32, pad)
    src3d = jnp.concatenate([src, ar % N]).reshape(NW, CH, CHUNK)
    dst3d = jnp.concatenate(
        [dst, N + ar % (N_ACC - N)]).reshape(NW, CH, CHUNK)
    z2 = jnp.zeros((ZR, D), jnp.float32)
    z1 = jnp.zeros((ZR,), jnp.float32)
    b_eff = b_agg2 + jnp.mean(group_encodings, axis=0)

    h = _tc_linear_relu(x, W_feat, b_feat)
    sums1, cnt0, cnt1 = _SC_AGG_CNT(h, src3d, dst3d, z2, z1)
    cnt = jnp.stack([cnt0, cnt1], axis=0)
    h1 = _tc_norm_linear_relu(sums1, cnt, W_agg1, b_agg1)
    sums2 = _SC_AGG(h1, src3d, dst3d, z2)
    out = _tc_final(sums2, cnt, W_agg2, b_eff, W_c1, b_c1, W_c2, b_c2)
    return out[:N]


# speedup vs baseline: 10.5670x; 1.0010x over previous
import jax
import jax.numpy as jnp
from jax import lax
from jax.experimental import pallas as pl
from jax.experimental.pallas import tpu as pltpu
from jax.experimental.pallas import tpu_sc as plsc

N = 10000
D = 128
NC = 2
NS = 16
NW = NC * NS
CHUNK = 128
CH = 80
E_PAD = NW * CH * CHUNK
N_ACC = 10240
ZR = N_ACC // NS
ROWBLK = 1280


def _lin_relu_body(x_ref, w_ref, b_ref, o_ref):
    o_ref[...] = jnp.maximum(
        jnp.dot(x_ref[...], w_ref[...], preferred_element_type=jnp.float32)
        + b_ref[...], 0.0)


def _tc_linear_relu(x, w, b):
    n = x.shape[0]
    blk = 1000
    return pl.pallas_call(
        _lin_relu_body,
        grid=(n // blk,),
        in_specs=[pl.BlockSpec((blk, D), lambda i: (i, 0)),
                  pl.BlockSpec((D, D), lambda i: (0, 0)),
                  pl.BlockSpec((1, D), lambda i: (0, 0))],
        out_specs=pl.BlockSpec((blk, D), lambda i: (i, 0)),
        out_shape=jax.ShapeDtypeStruct((n, D), jnp.float32),
    )(x, w, b.reshape(1, D))


def _norm_lin_relu_body(s_ref, c_ref, w_ref, b_ref, o_ref):
    cnt = c_ref[0] + c_ref[1]
    m = (s_ref[0] + s_ref[1]) / jnp.maximum(cnt, 1.0)
    o_ref[...] = jnp.maximum(
        jnp.dot(m, w_ref[...], preferred_element_type=jnp.float32)
        + b_ref[...], 0.0)


def _tc_norm_linear_relu(sums, cnt, w, b):
    return pl.pallas_call(
        _norm_lin_relu_body,
        grid=(N_ACC // ROWBLK,),
        in_specs=[pl.BlockSpec((NC, ROWBLK, D), lambda i: (0, i, 0)),
                  pl.BlockSpec((NC, ROWBLK, 1), lambda i: (0, i, 0)),
                  pl.BlockSpec((D, D), lambda i: (0, 0)),
                  pl.BlockSpec((1, D), lambda i: (0, 0))],
        out_specs=pl.BlockSpec((ROWBLK, D), lambda i: (i, 0)),
        out_shape=jax.ShapeDtypeStruct((N_ACC, D), jnp.float32),
    )(sums, cnt.reshape(NC, N_ACC, 1), w, b.reshape(1, D))


def _final_body(s_ref, c_ref, w2_ref, b2_ref, wc1_ref, bc1_ref, wc2_ref,
                bc2_ref, o_ref):
    cnt = c_ref[0] + c_ref[1]
    m = (s_ref[0] + s_ref[1]) / jnp.maximum(cnt, 1.0)
    h2 = jnp.dot(m, w2_ref[...], preferred_element_type=jnp.float32) + b2_ref[...]
    hid = jnp.maximum(
        jnp.dot(h2, wc1_ref[...], preferred_element_type=jnp.float32)
        + bc1_ref[...], 0.0)
    o_ref[...] = jnp.dot(hid, wc2_ref[...],
                         preferred_element_type=jnp.float32) + bc2_ref[...]


def _tc_final(sums, cnt, w2, b2, wc1, bc1, wc2, bc2):
    h2w = wc1.shape[0]
    hid = wc1.shape[1]
    out = wc2.shape[1]
    return pl.pallas_call(
        _final_body,
        grid=(N_ACC // ROWBLK,),
        in_specs=[pl.BlockSpec((NC, ROWBLK, D), lambda i: (0, i, 0)),
                  pl.BlockSpec((NC, ROWBLK, 1), lambda i: (0, i, 0)),
                  pl.BlockSpec((D, D), lambda i: (0, 0)),
                  pl.BlockSpec((1, D), lambda i: (0, 0)),
                  pl.BlockSpec((h2w, hid), lambda i: (0, 0)),
                  pl.BlockSpec((1, hid), lambda i: (0, 0)),
                  pl.BlockSpec((hid, out), lambda i: (0, 0)),
                  pl.BlockSpec((1, out), lambda i: (0, 0))],
        out_specs=pl.BlockSpec((ROWBLK, out), lambda i: (i, 0)),
        out_shape=jax.ShapeDtypeStruct((N_ACC, out), jnp.float32),
    )(sums, cnt.reshape(NC, N_ACC, 1), w2, b2.reshape(1, D),
      wc1, bc1.reshape(1, hid), wc2, bc2.reshape(1, out))


_MESH = plsc.VectorSubcoreMesh(core_axis_name="c", subcore_axis_name="s",
                               num_cores=NC, num_subcores=NS)


NBUF = 2
IH = 40
G_STEPS = IH // NBUF


def _agg_pipeline(h_hbm, acc, cacc, idxs, idxd, rows, ones, gsems, ssems,
                  csems, with_count):
    def start_gather(j, b):
        pltpu.async_copy(h_hbm.at[idxs.at[j]], rows.at[b], gsems.at[b])

    def wait_gather(b):
        pltpu.make_async_copy(h_hbm.at[idxs.at[0]], rows.at[b],
                              gsems.at[b]).wait()

    def start_scatter(j, b):
        pltpu.async_copy(rows.at[b], acc.at[idxd.at[j]], ssems.at[b],
                         add=True)
        if with_count:
            pltpu.async_copy(ones, cacc.at[idxd.at[j]], csems.at[b],
                             add=True)

    def wait_scatter(b):
        pltpu.make_async_copy(rows.at[b], acc.at[idxd.at[0]],
                              ssems.at[b]).wait()
        if with_count:
            pltpu.make_async_copy(ones, cacc.at[idxd.at[0]],
                                  csems.at[b]).wait()

    for b in range(NBUF):
        start_gather(b, b)

    def step(g, carry):
        for b in range(NBUF):
            wait_gather(b)
            start_scatter(g * NBUF + b, b)
        for b in range(NBUF):
            wait_scatter(b)
            start_gather((g + 1) * NBUF + b, b)
        return carry

    lax.fori_loop(0, G_STEPS - 1, step, 0)
    g = G_STEPS - 1
    for b in range(NBUF):
        wait_gather(b)
        start_scatter(g * NBUF + b, b)
    for b in range(NBUF):
        wait_scatter(b)


def _sc_agg_cnt_body(h_hbm, src_hbm, dst_hbm, z2_hbm, z1_hbm,
                     sums_out, cnt_out0, cnt_out1, acc, cacc, idxs, idxd,
                     rows, ones, gsems, ssems, csems):
    cid = lax.axis_index("c")
    sid = lax.axis_index("s")
    wid = sid * NC + cid
    zbase = sid * ZR
    pltpu.sync_copy(z2_hbm, acc.at[pl.ds(zbase, ZR)])
    pltpu.sync_copy(z1_hbm, cacc.at[pl.ds(zbase, ZR)])
    for i in range(CHUNK // 16):
        ones[pl.ds(i * 16, 16)] = jnp.ones((16,), jnp.float32)
    plsc.subcore_barrier()
    for half in range(CH // IH):
        pltpu.sync_copy(src_hbm.at[wid, pl.ds(half * IH, IH)], idxs)
        pltpu.sync_copy(dst_hbm.at[wid, pl.ds(half * IH, IH)], idxd)
        _agg_pipeline(h_hbm, acc, cacc, idxs, idxd, rows, ones, gsems,
                      ssems, csems, with_count=True)
    plsc.subcore_barrier()
    pltpu.sync_copy(acc.at[pl.ds(zbase, ZR)],
                    sums_out.at[cid, pl.ds(zbase, ZR)])

    @pl.when(cid == 0)
    def _():
        pltpu.sync_copy(cacc.at[pl.ds(zbase, ZR)],
                        cnt_out0.at[pl.ds(zbase, ZR)])

    @pl.when(cid == 1)
    def _():
        pltpu.sync_copy(cacc.at[pl.ds(zbase, ZR)],
                        cnt_out1.at[pl.ds(zbase, ZR)])


def _sc_agg_body(h_hbm, src_hbm, dst_hbm, z2_hbm,
                 sums_out, acc, idxs, idxd, rows, gsems, ssems):
    cid = lax.axis_index("c")
    sid = lax.axis_index("s")
    wid = sid * NC + cid
    zbase = sid * ZR
    pltpu.sync_copy(z2_hbm, acc.at[pl.ds(zbase, ZR)])
    plsc.subcore_barrier()
    for half in range(CH // IH):
        pltpu.sync_copy(src_hbm.at[wid, pl.ds(half * IH, IH)], idxs)
        pltpu.sync_copy(dst_hbm.at[wid, pl.ds(half * IH, IH)], idxd)
        _agg_pipeline(h_hbm, acc, None, idxs, idxd, rows, None, gsems,
                      ssems, None, with_count=False)
    plsc.subcore_barrier()
    pltpu.sync_copy(acc.at[pl.ds(zbase, ZR)],
                    sums_out.at[cid, pl.ds(zbase, ZR)])


_SC_AGG_CNT = pl.kernel(
    _sc_agg_cnt_body,
    out_type=(jax.ShapeDtypeStruct((NC, N_ACC, D), jnp.float32),
              jax.ShapeDtypeStruct((N_ACC,), jnp.float32),
              jax.ShapeDtypeStruct((N_ACC,), jnp.float32)),
    mesh=_MESH,
    compiler_params=pltpu.CompilerParams(use_tc_tiling_on_sc=True),
    scratch_types=[
        pltpu.VMEM_SHARED((N_ACC, D), jnp.float32),
        pltpu.VMEM_SHARED((N_ACC,), jnp.float32),
        pltpu.VMEM((IH, CHUNK), jnp.int32),
        pltpu.VMEM((IH, CHUNK), jnp.int32),
        pltpu.VMEM((NBUF, CHUNK, D), jnp.float32),
        pltpu.VMEM((CHUNK,), jnp.float32),
        pltpu.SemaphoreType.DMA((NBUF,)),
        pltpu.SemaphoreType.DMA((NBUF,)),
        pltpu.SemaphoreType.DMA((NBUF,)),
    ],
)

_SC_AGG = pl.kernel(
    _sc_agg_body,
    out_type=jax.ShapeDtypeStruct((NC, N_ACC, D), jnp.float32),
    mesh=_MESH,
    compiler_params=pltpu.CompilerParams(use_tc_tiling_on_sc=True),
    scratch_types=[
        pltpu.VMEM_SHARED((N_ACC, D), jnp.float32),
        pltpu.VMEM((IH, CHUNK), jnp.int32),
        pltpu.VMEM((IH, CHUNK), jnp.int32),
        pltpu.VMEM((NBUF, CHUNK, D), jnp.float32),
        pltpu.SemaphoreType.DMA((NBUF,)),
        pltpu.SemaphoreType.DMA((NBUF,)),
    ],
)


def kernel(x, edge_index, W_feat, b_feat, group_encodings, W_agg1, b_agg1,
           W_agg2, b_agg2, W_c1, b_c1, W_c2, b_c2):
    src = edge_index[0]
    dst = edge_index[1]
    e = src.shape[0]
    pad = E_PAD - e
    ar = lax.iota(jnp.int32, pad)
    src3d = jnp.concatenate([src, ar % N]).reshape(NW, CH, CHUNK)
    dst3d = jnp.concatenate(
        [dst, N + ar % (N_ACC - N)]).reshape(NW, CH, CHUNK)
    z2 = jnp.zeros((ZR, D), jnp.float32)
    z1 = jnp.zeros((ZR,), jnp.float32)
    b_eff = b_agg2 + jnp.mean(group_encodings, axis=0)

    h = _tc_linear_relu(x, W_feat, b_feat)
    sums1, cnt0, cnt1 = _SC_AGG_CNT(h, src3d, dst3d, z2, z1)
    cnt = jnp.stack([cnt0, cnt1], axis=0)
    h1 = _tc_norm_linear_relu(sums1, cnt, W_agg1, b_agg1)
    sums2 = _SC_AGG(h1, src3d, dst3d, z2)
    out = _tc_final(sums2, cnt, W_agg2, b_eff, W_c1, b_c1, W_c2, b_c2)
    return out[:N]


# speedup vs baseline: 11.9644x; 1.1322x over previous
import jax
import jax.numpy as jnp
from jax import lax
from jax.experimental import pallas as pl
from jax.experimental.pallas import tpu as pltpu
from jax.experimental.pallas import tpu_sc as plsc

N = 10000
D = 128
NC = 2
NS = 16
NW = NC * NS
CHUNK = 128
CH = 80
E_PAD = NW * CH * CHUNK
N_ACC = 10240
ZR = N_ACC // NS
ROWBLK = 1280


def _lin_relu_body(x_ref, w_ref, b_ref, o_ref):
    o_ref[...] = jnp.maximum(
        jnp.dot(x_ref[...], w_ref[...], preferred_element_type=jnp.float32)
        + b_ref[...], 0.0)


def _tc_linear_relu(x, w, b):
    n = x.shape[0]
    blk = 1000
    return pl.pallas_call(
        _lin_relu_body,
        grid=(n // blk,),
        in_specs=[pl.BlockSpec((blk, D), lambda i: (i, 0)),
                  pl.BlockSpec((D, D), lambda i: (0, 0)),
                  pl.BlockSpec((1, D), lambda i: (0, 0))],
        out_specs=pl.BlockSpec((blk, D), lambda i: (i, 0)),
        out_shape=jax.ShapeDtypeStruct((n, D), jnp.float32),
    )(x, w, b.reshape(1, D))


def _norm_lin_relu_body(s_ref, c_ref, w_ref, b_ref, o_ref):
    cnt = c_ref[0] + c_ref[1]
    m = (s_ref[0] + s_ref[1]) / jnp.maximum(cnt, 1.0)
    o_ref[...] = jnp.maximum(
        jnp.dot(m, w_ref[...], preferred_element_type=jnp.float32)
        + b_ref[...], 0.0)


def _tc_norm_linear_relu(sums, cnt, w, b):
    return pl.pallas_call(
        _norm_lin_relu_body,
        grid=(N_ACC // ROWBLK,),
        in_specs=[pl.BlockSpec((NC, ROWBLK, D), lambda i: (0, i, 0)),
                  pl.BlockSpec((NC, ROWBLK, 1), lambda i: (0, i, 0)),
                  pl.BlockSpec((D, D), lambda i: (0, 0)),
                  pl.BlockSpec((1, D), lambda i: (0, 0))],
        out_specs=pl.BlockSpec((ROWBLK, D), lambda i: (i, 0)),
        out_shape=jax.ShapeDtypeStruct((N_ACC, D), jnp.float32),
    )(sums, cnt.reshape(NC, N_ACC, 1), w, b.reshape(1, D))


def _norm_lin_q_body(s_ref, c_ref, w1_ref, b1_ref, w2_ref, wc1_ref, o_ref):
    cnt = c_ref[0] + c_ref[1]
    m = (s_ref[0] + s_ref[1]) / jnp.maximum(cnt, 1.0)
    h1 = jnp.maximum(
        jnp.dot(m, w1_ref[...], preferred_element_type=jnp.float32)
        + b1_ref[...], 0.0)
    wq = jnp.dot(w2_ref[...], wc1_ref[...],
                 preferred_element_type=jnp.float32)
    o_ref[...] = jnp.dot(h1, wq, preferred_element_type=jnp.float32)


def _tc_norm_lin_q(sums, cnt, w1, b1, w2, wc1):
    hid = wc1.shape[1]
    return pl.pallas_call(
        _norm_lin_q_body,
        grid=(N_ACC // ROWBLK,),
        in_specs=[pl.BlockSpec((NC, ROWBLK, D), lambda i: (0, i, 0)),
                  pl.BlockSpec((NC, ROWBLK, 1), lambda i: (0, i, 0)),
                  pl.BlockSpec((D, D), lambda i: (0, 0)),
                  pl.BlockSpec((1, D), lambda i: (0, 0)),
                  pl.BlockSpec((D, D), lambda i: (0, 0)),
                  pl.BlockSpec((D, hid), lambda i: (0, 0))],
        out_specs=pl.BlockSpec((ROWBLK, hid), lambda i: (i, 0)),
        out_shape=jax.ShapeDtypeStruct((N_ACC, hid), jnp.float32),
    )(sums, cnt.reshape(NC, N_ACC, 1), w1, b1.reshape(1, D), w2, wc1)


def _final_body(s_ref, c_ref, beff_ref, wc1_ref, bc1_ref, wc2_ref,
                bc2_ref, o_ref):
    cnt = c_ref[0] + c_ref[1]
    m = (s_ref[0] + s_ref[1]) / jnp.maximum(cnt, 1.0)
    cb = jnp.dot(beff_ref[...], wc1_ref[...],
                 preferred_element_type=jnp.float32) + bc1_ref[...]
    hid = jnp.maximum(m + cb, 0.0)
    o_ref[...] = jnp.dot(hid, wc2_ref[...],
                         preferred_element_type=jnp.float32) + bc2_ref[...]


def _tc_final(sums_q, cnt, b_eff, wc1, bc1, wc2, bc2):
    hid = wc1.shape[1]
    out = wc2.shape[1]
    return pl.pallas_call(
        _final_body,
        grid=(N_ACC // ROWBLK,),
        in_specs=[pl.BlockSpec((NC, ROWBLK, hid), lambda i: (0, i, 0)),
                  pl.BlockSpec((NC, ROWBLK, 1), lambda i: (0, i, 0)),
                  pl.BlockSpec((1, D), lambda i: (0, 0)),
                  pl.BlockSpec((D, hid), lambda i: (0, 0)),
                  pl.BlockSpec((1, hid), lambda i: (0, 0)),
                  pl.BlockSpec((hid, out), lambda i: (0, 0)),
                  pl.BlockSpec((1, out), lambda i: (0, 0))],
        out_specs=pl.BlockSpec((ROWBLK, out), lambda i: (i, 0)),
        out_shape=jax.ShapeDtypeStruct((N_ACC, out), jnp.float32),
    )(sums_q, cnt.reshape(NC, N_ACC, 1), b_eff.reshape(1, D),
      wc1, bc1.reshape(1, hid), wc2, bc2.reshape(1, out))


_MESH = plsc.VectorSubcoreMesh(core_axis_name="c", subcore_axis_name="s",
                               num_cores=NC, num_subcores=NS)


NBUF = 2
IH = 40
G_STEPS = IH // NBUF


def _agg_pipeline(h_hbm, acc, cacc, idxs, idxd, rows, ones, gsems, ssems,
                  csems, with_count):
    def start_gather(j, b):
        pltpu.async_copy(h_hbm.at[idxs.at[j]], rows.at[b], gsems.at[b])

    def wait_gather(b):
        pltpu.make_async_copy(h_hbm.at[idxs.at[0]], rows.at[b],
                              gsems.at[b]).wait()

    def start_scatter(j, b):
        pltpu.async_copy(rows.at[b], acc.at[idxd.at[j]], ssems.at[b],
                         add=True)
        if with_count:
            pltpu.async_copy(ones, cacc.at[idxd.at[j]], csems.at[b],
                             add=True)

    def wait_scatter(b):
        pltpu.make_async_copy(rows.at[b], acc.at[idxd.at[0]],
                              ssems.at[b]).wait()
        if with_count:
            pltpu.make_async_copy(ones, cacc.at[idxd.at[0]],
                                  csems.at[b]).wait()

    for b in range(NBUF):
        start_gather(b, b)

    def step(g, carry):
        for b in range(NBUF):
            wait_gather(b)
            start_scatter(g * NBUF + b, b)
        for b in range(NBUF):
            wait_scatter(b)
            start_gather((g + 1) * NBUF + b, b)
        return carry

    lax.fori_loop(0, G_STEPS - 1, step, 0)
    g = G_STEPS - 1
    for b in range(NBUF):
        wait_gather(b)
        start_scatter(g * NBUF + b, b)
    for b in range(NBUF):
        wait_scatter(b)


def _sc_agg_cnt_body(h_hbm, src_hbm, dst_hbm, z2_hbm, z1_hbm,
                     sums_out, cnt_out0, cnt_out1, acc, cacc, idxs, idxd,
                     rows, ones, gsems, ssems, csems):
    cid = lax.axis_index("c")
    sid = lax.axis_index("s")
    wid = sid * NC + cid
    zbase = sid * ZR
    pltpu.sync_copy(z2_hbm, acc.at[pl.ds(zbase, ZR)])
    pltpu.sync_copy(z1_hbm, cacc.at[pl.ds(zbase, ZR)])
    for i in range(CHUNK // 16):
        ones[pl.ds(i * 16, 16)] = jnp.ones((16,), jnp.float32)
    plsc.subcore_barrier()
    for half in range(CH // IH):
        pltpu.sync_copy(src_hbm.at[wid, pl.ds(half * IH, IH)], idxs)
        pltpu.sync_copy(dst_hbm.at[wid, pl.ds(half * IH, IH)], idxd)
        _agg_pipeline(h_hbm, acc, cacc, idxs, idxd, rows, ones, gsems,
                      ssems, csems, with_count=True)
    plsc.subcore_barrier()
    pltpu.sync_copy(acc.at[pl.ds(zbase, ZR)],
                    sums_out.at[cid, pl.ds(zbase, ZR)])

    @pl.when(cid == 0)
    def _():
        pltpu.sync_copy(cacc.at[pl.ds(zbase, ZR)],
                        cnt_out0.at[pl.ds(zbase, ZR)])

    @pl.when(cid == 1)
    def _():
        pltpu.sync_copy(cacc.at[pl.ds(zbase, ZR)],
                        cnt_out1.at[pl.ds(zbase, ZR)])


def _sc_agg_body(h_hbm, src_hbm, dst_hbm, z2_hbm,
                 sums_out, acc, idxs, idxd, rows, gsems, ssems):
    cid = lax.axis_index("c")
    sid = lax.axis_index("s")
    wid = sid * NC + cid
    zbase = sid * ZR
    pltpu.sync_copy(z2_hbm, acc.at[pl.ds(zbase, ZR)])
    plsc.subcore_barrier()
    for half in range(CH // IH):
        pltpu.sync_copy(src_hbm.at[wid, pl.ds(half * IH, IH)], idxs)
        pltpu.sync_copy(dst_hbm.at[wid, pl.ds(half * IH, IH)], idxd)
        _agg_pipeline(h_hbm, acc, None, idxs, idxd, rows, None, gsems,
                      ssems, None, with_count=False)
    plsc.subcore_barrier()
    pltpu.sync_copy(acc.at[pl.ds(zbase, ZR)],
                    sums_out.at[cid, pl.ds(zbase, ZR)])


_SC_AGG_CNT = pl.kernel(
    _sc_agg_cnt_body,
    out_type=(jax.ShapeDtypeStruct((NC, N_ACC, D), jnp.float32),
              jax.ShapeDtypeStruct((N_ACC,), jnp.float32),
              jax.ShapeDtypeStruct((N_ACC,), jnp.float32)),
    mesh=_MESH,
    compiler_params=pltpu.CompilerParams(use_tc_tiling_on_sc=True),
    scratch_types=[
        pltpu.VMEM_SHARED((N_ACC, D), jnp.float32),
        pltpu.VMEM_SHARED((N_ACC,), jnp.float32),
        pltpu.VMEM((IH, CHUNK), jnp.int32),
        pltpu.VMEM((IH, CHUNK), jnp.int32),
        pltpu.VMEM((NBUF, CHUNK, D), jnp.float32),
        pltpu.VMEM((CHUNK,), jnp.float32),
        pltpu.SemaphoreType.DMA((NBUF,)),
        pltpu.SemaphoreType.DMA((NBUF,)),
        pltpu.SemaphoreType.DMA((NBUF,)),
    ],
)

def _sc_agg_sp_body(q_hbm, src_hbm, dst_hbm, z64_hbm, sums_out,
                    acc, qsp, idxs, idxd, rows, gsems, ssems):
    cid = lax.axis_index("c")
    sid = lax.axis_index("s")
    wid = sid * NC + cid
    zbase = sid * ZR
    pltpu.sync_copy(z64_hbm, acc.at[pl.ds(zbase, ZR)])
    pltpu.sync_copy(q_hbm.at[pl.ds(zbase, ZR)], qsp.at[pl.ds(zbase, ZR)])
    plsc.subcore_barrier()
    for half in range(CH // IH):
        pltpu.sync_copy(src_hbm.at[wid, pl.ds(half * IH, IH)], idxs)
        pltpu.sync_copy(dst_hbm.at[wid, pl.ds(half * IH, IH)], idxd)
        _agg_pipeline(qsp, acc, None, idxs, idxd, rows, None, gsems,
                      ssems, None, with_count=False)
    plsc.subcore_barrier()
    pltpu.sync_copy(acc.at[pl.ds(zbase, ZR)],
                    sums_out.at[cid, pl.ds(zbase, ZR)])


_SC_AGG64 = pl.kernel(
    _sc_agg_sp_body,
    out_type=jax.ShapeDtypeStruct((NC, N_ACC, 64), jnp.float32),
    mesh=_MESH,
    scratch_types=[
        pltpu.VMEM_SHARED((N_ACC, 64), jnp.float32),
        pltpu.VMEM_SHARED((N_ACC, 64), jnp.float32),
        pltpu.VMEM((IH, CHUNK), jnp.int32),
        pltpu.VMEM((IH, CHUNK), jnp.int32),
        pltpu.VMEM((NBUF, CHUNK, 64), jnp.float32),
        pltpu.SemaphoreType.DMA((NBUF,)),
        pltpu.SemaphoreType.DMA((NBUF,)),
    ],
)


def kernel(x, edge_index, W_feat, b_feat, group_encodings, W_agg1, b_agg1,
           W_agg2, b_agg2, W_c1, b_c1, W_c2, b_c2):
    src = edge_index[0]
    dst = edge_index[1]
    e = src.shape[0]
    pad = E_PAD - e
    ar = lax.iota(jnp.int32, pad)
    src3d = jnp.concatenate([src, ar % N]).reshape(NW, CH, CHUNK)
    dst3d = jnp.concatenate(
        [dst, N + ar % (N_ACC - N)]).reshape(NW, CH, CHUNK)
    z2 = jnp.zeros((ZR, D), jnp.float32)
    z1 = jnp.zeros((ZR,), jnp.float32)
    z64 = jnp.zeros((ZR, 64), jnp.float32)
    b_eff = b_agg2 + jnp.mean(group_encodings, axis=0)

    h = _tc_linear_relu(x, W_feat, b_feat)
    sums1, cnt0, cnt1 = _SC_AGG_CNT(h, src3d, dst3d, z2, z1)
    cnt = jnp.stack([cnt0, cnt1], axis=0)
    q = _tc_norm_lin_q(sums1, cnt, W_agg1, b_agg1, W_agg2, W_c1)
    sums_q = _SC_AGG64(q, src3d, dst3d, z64)
    out = _tc_final(sums_q, cnt, b_eff, W_c1, b_c1, W_c2, b_c2)
    return out[:N]


# speedup vs baseline: 12.0240x; 1.0050x over previous
import jax
import jax.numpy as jnp
from jax import lax
from jax.experimental import pallas as pl
from jax.experimental.pallas import tpu as pltpu
from jax.experimental.pallas import tpu_sc as plsc

N = 10000
D = 128
NC = 2
NS = 16
NW = NC * NS
CHUNK = 128
CH = 80
E_PAD = NW * CH * CHUNK
N_ACC = 10240
ZR = N_ACC // NS
ROWBLK = 1280


def _lin_relu_body(x_ref, w_ref, b_ref, o_ref):
    o_ref[...] = jnp.maximum(
        jnp.dot(x_ref[...], w_ref[...], preferred_element_type=jnp.float32)
        + b_ref[...], 0.0)


def _tc_linear_relu(x, w, b):
    n = x.shape[0]
    blk = 1000
    return pl.pallas_call(
        _lin_relu_body,
        grid=(n // blk,),
        in_specs=[pl.BlockSpec((blk, D), lambda i: (i, 0)),
                  pl.BlockSpec((D, D), lambda i: (0, 0)),
                  pl.BlockSpec((1, D), lambda i: (0, 0))],
        out_specs=pl.BlockSpec((blk, D), lambda i: (i, 0)),
        out_shape=jax.ShapeDtypeStruct((n, D), jnp.float32),
    )(x, w, b.reshape(1, D))


def _norm_lin_relu_body(s_ref, c_ref, w_ref, b_ref, o_ref):
    cnt = c_ref[0] + c_ref[1]
    m = (s_ref[0] + s_ref[1]) / jnp.maximum(cnt, 1.0)
    o_ref[...] = jnp.maximum(
        jnp.dot(m, w_ref[...], preferred_element_type=jnp.float32)
        + b_ref[...], 0.0)


def _tc_norm_linear_relu(sums, cnt, w, b):
    return pl.pallas_call(
        _norm_lin_relu_body,
        grid=(N_ACC // ROWBLK,),
        in_specs=[pl.BlockSpec((NC, ROWBLK, D), lambda i: (0, i, 0)),
                  pl.BlockSpec((NC, ROWBLK, 1), lambda i: (0, i, 0)),
                  pl.BlockSpec((D, D), lambda i: (0, 0)),
                  pl.BlockSpec((1, D), lambda i: (0, 0))],
        out_specs=pl.BlockSpec((ROWBLK, D), lambda i: (i, 0)),
        out_shape=jax.ShapeDtypeStruct((N_ACC, D), jnp.float32),
    )(sums, cnt.reshape(NC, N_ACC, 1), w, b.reshape(1, D))


def _norm_lin_q_body(s_ref, c_ref, w1_ref, b1_ref, w2_ref, wc1_ref, o_ref):
    cnt = c_ref[0] + c_ref[1]
    m = (s_ref[0] + s_ref[1]) / jnp.maximum(cnt, 1.0)
    h1 = jnp.maximum(
        jnp.dot(m, w1_ref[...], preferred_element_type=jnp.float32)
        + b1_ref[...], 0.0)
    wq = jnp.dot(w2_ref[...], wc1_ref[...],
                 preferred_element_type=jnp.float32,
                 precision=lax.Precision.HIGHEST)
    o_ref[...] = jnp.dot(h1, wq, preferred_element_type=jnp.float32,
                         precision=lax.Precision.HIGHEST)


def _tc_norm_lin_q(sums, cnt, w1, b1, w2, wc1):
    hid = wc1.shape[1]
    return pl.pallas_call(
        _norm_lin_q_body,
        grid=(N_ACC // ROWBLK,),
        in_specs=[pl.BlockSpec((NC, ROWBLK, D), lambda i: (0, i, 0)),
                  pl.BlockSpec((NC, ROWBLK, 1), lambda i: (0, i, 0)),
                  pl.BlockSpec((D, D), lambda i: (0, 0)),
                  pl.BlockSpec((1, D), lambda i: (0, 0)),
                  pl.BlockSpec((D, D), lambda i: (0, 0)),
                  pl.BlockSpec((D, hid), lambda i: (0, 0))],
        out_specs=pl.BlockSpec((ROWBLK, hid), lambda i: (i, 0)),
        out_shape=jax.ShapeDtypeStruct((N_ACC, hid), jnp.float32),
    )(sums, cnt.reshape(NC, N_ACC, 1), w1, b1.reshape(1, D), w2, wc1)


def _final_body(s_ref, c_ref, beff_ref, wc1_ref, bc1_ref, wc2_ref,
                bc2_ref, o_ref):
    cnt = c_ref[0] + c_ref[1]
    m = (s_ref[0] + s_ref[1]) / jnp.maximum(cnt, 1.0)
    cb = jnp.dot(beff_ref[...], wc1_ref[...],
                 preferred_element_type=jnp.float32,
                 precision=lax.Precision.HIGHEST) + bc1_ref[...]
    hid = jnp.maximum(m + cb, 0.0)
    o_ref[...] = jnp.dot(hid, wc2_ref[...],
                         preferred_element_type=jnp.float32) + bc2_ref[...]


def _tc_final(sums_q, cnt, b_eff, wc1, bc1, wc2, bc2):
    hid = wc1.shape[1]
    out = wc2.shape[1]
    return pl.pallas_call(
        _final_body,
        grid=(N_ACC // ROWBLK,),
        in_specs=[pl.BlockSpec((NC, ROWBLK, hid), lambda i: (0, i, 0)),
                  pl.BlockSpec((NC, ROWBLK, 1), lambda i: (0, i, 0)),
                  pl.BlockSpec((1, D), lambda i: (0, 0)),
                  pl.BlockSpec((D, hid), lambda i: (0, 0)),
                  pl.BlockSpec((1, hid), lambda i: (0, 0)),
                  pl.BlockSpec((hid, out), lambda i: (0, 0)),
                  pl.BlockSpec((1, out), lambda i: (0, 0))],
        out_specs=pl.BlockSpec((ROWBLK, out), lambda i: (i, 0)),
        out_shape=jax.ShapeDtypeStruct((N_ACC, out), jnp.float32),
    )(sums_q, cnt.reshape(NC, N_ACC, 1), b_eff.reshape(1, D),
      wc1, bc1.reshape(1, hid), wc2, bc2.reshape(1, out))


_MESH = plsc.VectorSubcoreMesh(core_axis_name="c", subcore_axis_name="s",
                               num_cores=NC, num_subcores=NS)


NBUF = 2
IH = 40
G_STEPS = IH // NBUF


def _agg_pipeline(h_hbm, acc, cacc, idxs, idxd, rows, ones, gsems, ssems,
                  csems, with_count):
    def start_gather(j, b):
        pltpu.async_copy(h_hbm.at[idxs.at[j]], rows.at[b], gsems.at[b])

    def wait_gather(b):
        pltpu.make_async_copy(h_hbm.at[idxs.at[0]], rows.at[b],
                              gsems.at[b]).wait()

    def start_scatter(j, b):
        pltpu.async_copy(rows.at[b], acc.at[idxd.at[j]], ssems.at[b],
                         add=True)
        if with_count:
            pltpu.async_copy(ones, cacc.at[idxd.at[j]], csems.at[b],
                             add=True)

    def wait_scatter(b):
        pltpu.make_async_copy(rows.at[b], acc.at[idxd.at[0]],
                              ssems.at[b]).wait()
        if with_count:
            pltpu.make_async_copy(ones, cacc.at[idxd.at[0]],
                                  csems.at[b]).wait()

    for b in range(NBUF):
        start_gather(b, b)

    def step(g, carry):
        for b in range(NBUF):
            wait_gather(b)
            start_scatter(g * NBUF + b, b)
        for b in range(NBUF):
            wait_scatter(b)
            start_gather((g + 1) * NBUF + b, b)
        return carry

    lax.fori_loop(0, G_STEPS - 1, step, 0)
    g = G_STEPS - 1
    for b in range(NBUF):
        wait_gather(b)
        start_scatter(g * NBUF + b, b)
    for b in range(NBUF):
        wait_scatter(b)


def _sc_agg_cnt_body(h_hbm, src_hbm, dst_hbm, z2_hbm, z1_hbm,
                     sums_out, cnt_out0, cnt_out1, acc, cacc, idxs, idxd,
                     rows, ones, gsems, ssems, csems):
    cid = lax.axis_index("c")
    sid = lax.axis_index("s")
    wid = sid * NC + cid
    zbase = sid * ZR
    pltpu.sync_copy(z2_hbm, acc.at[pl.ds(zbase, ZR)])
    pltpu.sync_copy(z1_hbm, cacc.at[pl.ds(zbase, ZR)])
    for i in range(CHUNK // 16):
        ones[pl.ds(i * 16, 16)] = jnp.ones((16,), jnp.float32)
    plsc.subcore_barrier()
    for half in range(CH // IH):
        pltpu.sync_copy(src_hbm.at[wid, pl.ds(half * IH, IH)], idxs)
        pltpu.sync_copy(dst_hbm.at[wid, pl.ds(half * IH, IH)], idxd)
        _agg_pipeline(h_hbm, acc, cacc, idxs, idxd, rows, ones, gsems,
                      ssems, csems, with_count=True)
    plsc.subcore_barrier()
    pltpu.sync_copy(acc.at[pl.ds(zbase, ZR)],
                    sums_out.at[cid, pl.ds(zbase, ZR)])

    @pl.when(cid == 0)
    def _():
        pltpu.sync_copy(cacc.at[pl.ds(zbase, ZR)],
                        cnt_out0.at[pl.ds(zbase, ZR)])

    @pl.when(cid == 1)
    def _():
        pltpu.sync_copy(cacc.at[pl.ds(zbase, ZR)],
                        cnt_out1.at[pl.ds(zbase, ZR)])


def _sc_agg_body(h_hbm, src_hbm, dst_hbm, z2_hbm,
                 sums_out, acc, idxs, idxd, rows, gsems, ssems):
    cid = lax.axis_index("c")
    sid = lax.axis_index("s")
    wid = sid * NC + cid
    zbase = sid * ZR
    pltpu.sync_copy(z2_hbm, acc.at[pl.ds(zbase, ZR)])
    plsc.subcore_barrier()
    for half in range(CH // IH):
        pltpu.sync_copy(src_hbm.at[wid, pl.ds(half * IH, IH)], idxs)
        pltpu.sync_copy(dst_hbm.at[wid, pl.ds(half * IH, IH)], idxd)
        _agg_pipeline(h_hbm, acc, None, idxs, idxd, rows, None, gsems,
                      ssems, None, with_count=False)
    plsc.subcore_barrier()
    pltpu.sync_copy(acc.at[pl.ds(zbase, ZR)],
                    sums_out.at[cid, pl.ds(zbase, ZR)])


_SC_AGG_CNT = pl.kernel(
    _sc_agg_cnt_body,
    out_type=(jax.ShapeDtypeStruct((NC, N_ACC, D), jnp.float32),
              jax.ShapeDtypeStruct((N_ACC,), jnp.float32),
              jax.ShapeDtypeStruct((N_ACC,), jnp.float32)),
    mesh=_MESH,
    compiler_params=pltpu.CompilerParams(use_tc_tiling_on_sc=True),
    scratch_types=[
        pltpu.VMEM_SHARED((N_ACC, D), jnp.float32),
        pltpu.VMEM_SHARED((N_ACC,), jnp.float32),
        pltpu.VMEM((IH, CHUNK), jnp.int32),
        pltpu.VMEM((IH, CHUNK), jnp.int32),
        pltpu.VMEM((NBUF, CHUNK, D), jnp.float32),
        pltpu.VMEM((CHUNK,), jnp.float32),
        pltpu.SemaphoreType.DMA((NBUF,)),
        pltpu.SemaphoreType.DMA((NBUF,)),
        pltpu.SemaphoreType.DMA((NBUF,)),
    ],
)

_SC_AGG64 = pl.kernel(
    _sc_agg_body,
    out_type=jax.ShapeDtypeStruct((NC, N_ACC, 64), jnp.float32),
    mesh=_MESH,
    compiler_params=pltpu.CompilerParams(use_tc_tiling_on_sc=False),
    scratch_types=[
        pltpu.VMEM_SHARED((N_ACC, 64), jnp.float32),
        pltpu.VMEM((IH, CHUNK), jnp.int32),
        pltpu.VMEM((IH, CHUNK), jnp.int32),
        pltpu.VMEM((NBUF, CHUNK, 64), jnp.float32),
        pltpu.SemaphoreType.DMA((NBUF,)),
        pltpu.SemaphoreType.DMA((NBUF,)),
    ],
)


def kernel(x, edge_index, W_feat, b_feat, group_encodings, W_agg1, b_agg1,
           W_agg2, b_agg2, W_c1, b_c1, W_c2, b_c2):
    src = edge_index[0]
    dst = edge_index[1]
    e = src.shape[0]
    pad = E_PAD - e
    ar = lax.iota(jnp.int32, pad)
    src3d = jnp.concatenate([src, ar % N]).reshape(NW, CH, CHUNK)
    dst3d = jnp.concatenate(
        [dst, N + ar % (N_ACC - N)]).reshape(NW, CH, CHUNK)
    z2 = jnp.zeros((ZR, D), jnp.float32)
    z1 = jnp.zeros((ZR,), jnp.float32)
    z64 = jnp.zeros((ZR, 64), jnp.float32)
    b_eff = b_agg2 + jnp.mean(group_encodings, axis=0)

    h = _tc_linear_relu(x, W_feat, b_feat)
    sums1, cnt0, cnt1 = _SC_AGG_CNT(h, src3d, dst3d, z2, z1)
    cnt = jnp.stack([cnt0, cnt1], axis=0)
    q = _tc_norm_lin_q(sums1, cnt, W_agg1, b_agg1, W_agg2, W_c1)
    sums_q = _SC_AGG64(q, src3d, dst3d, z64)
    out = _tc_final(sums_q, cnt, b_eff, W_c1, b_c1, W_c2, b_c2)
    return out[:N]


# speedup vs baseline: 12.4242x; 1.0333x over previous
import jax
import jax.numpy as jnp
from jax import lax
from jax.experimental import pallas as pl
from jax.experimental.pallas import tpu as pltpu
from jax.experimental.pallas import tpu_sc as plsc

N = 10000
D = 128
NC = 2
NS = 16
NW = NC * NS
CHUNK = 128
CH = 80
E_PAD = NW * CH * CHUNK
N_ACC = 10240
ZR = N_ACC // NS
ROWBLK = 1280


def _lin_relu_body(x_ref, w_ref, b_ref, o_ref):
    o_ref[...] = jnp.maximum(
        jnp.dot(x_ref[...], w_ref[...], preferred_element_type=jnp.float32)
        + b_ref[...], 0.0)


def _tc_linear_relu(x, w, b):
    n = x.shape[0]
    blk = 1000
    return pl.pallas_call(
        _lin_relu_body,
        grid=(n // blk,),
        in_specs=[pl.BlockSpec((blk, D), lambda i: (i, 0)),
                  pl.BlockSpec((D, D), lambda i: (0, 0)),
                  pl.BlockSpec((1, D), lambda i: (0, 0))],
        out_specs=pl.BlockSpec((blk, D), lambda i: (i, 0)),
        out_shape=jax.ShapeDtypeStruct((n, D), jnp.float32),
    )(x, w, b.reshape(1, D))


def _norm_lin_relu_body(s_ref, c_ref, w_ref, b_ref, o_ref):
    cnt = c_ref[0] + c_ref[1]
    m = (s_ref[0] + s_ref[1]) / jnp.maximum(cnt, 1.0)
    o_ref[...] = jnp.maximum(
        jnp.dot(m, w_ref[...], preferred_element_type=jnp.float32)
        + b_ref[...], 0.0)


def _tc_norm_linear_relu(sums, cnt, w, b):
    return pl.pallas_call(
        _norm_lin_relu_body,
        grid=(N_ACC // ROWBLK,),
        in_specs=[pl.BlockSpec((NC, ROWBLK, D), lambda i: (0, i, 0)),
                  pl.BlockSpec((NC, ROWBLK, 1), lambda i: (0, i, 0)),
                  pl.BlockSpec((D, D), lambda i: (0, 0)),
                  pl.BlockSpec((1, D), lambda i: (0, 0))],
        out_specs=pl.BlockSpec((ROWBLK, D), lambda i: (i, 0)),
        out_shape=jax.ShapeDtypeStruct((N_ACC, D), jnp.float32),
    )(sums, cnt.reshape(NC, N_ACC, 1), w, b.reshape(1, D))


def _norm_lin_q_body(s_ref, c_ref, w1_ref, b1_ref, w2_ref, wc1_ref, o_ref):
    cnt = c_ref[0] + c_ref[1]
    m = (s_ref[0] + s_ref[1]) / jnp.maximum(cnt, 1.0)
    h1 = jnp.maximum(
        jnp.dot(m, w1_ref[...], preferred_element_type=jnp.float32)
        + b1_ref[...], 0.0)
    wq = jnp.dot(w2_ref[...], wc1_ref[...],
                 preferred_element_type=jnp.float32,
                 precision=lax.Precision.HIGHEST)
    o_ref[...] = jnp.dot(h1, wq, preferred_element_type=jnp.float32,
                         precision=lax.Precision.HIGHEST)


def _tc_norm_lin_q(sums, cnt, w1, b1, w2, wc1):
    hid = wc1.shape[1]
    return pl.pallas_call(
        _norm_lin_q_body,
        grid=(N_ACC // ROWBLK,),
        in_specs=[pl.BlockSpec((NC, ROWBLK, D), lambda i: (0, i, 0)),
                  pl.BlockSpec((NC, ROWBLK, 1), lambda i: (0, i, 0)),
                  pl.BlockSpec((D, D), lambda i: (0, 0)),
                  pl.BlockSpec((1, D), lambda i: (0, 0)),
                  pl.BlockSpec((D, D), lambda i: (0, 0)),
                  pl.BlockSpec((D, hid), lambda i: (0, 0))],
        out_specs=pl.BlockSpec((ROWBLK, hid), lambda i: (i, 0)),
        out_shape=jax.ShapeDtypeStruct((N_ACC, hid), jnp.float32),
    )(sums, cnt.reshape(NC, N_ACC, 1), w1, b1.reshape(1, D), w2, wc1)


def _final_body(s_ref, c_ref, beff_ref, wc1_ref, bc1_ref, wc2_ref,
                bc2_ref, o_ref):
    cnt = c_ref[0] + c_ref[1]
    m = (s_ref[:, 0:64] + s_ref[:, 64:128]) / jnp.maximum(cnt, 1.0)
    cb = jnp.dot(beff_ref[...], wc1_ref[...],
                 preferred_element_type=jnp.float32,
                 precision=lax.Precision.HIGHEST) + bc1_ref[...]
    hid = jnp.maximum(m + cb, 0.0)
    o_ref[...] = jnp.dot(hid, wc2_ref[...],
                         preferred_element_type=jnp.float32) + bc2_ref[...]


def _tc_final(sums_q, cnt, b_eff, wc1, bc1, wc2, bc2):
    hid = wc1.shape[1]
    out = wc2.shape[1]
    blk = 1000
    return pl.pallas_call(
        _final_body,
        grid=(N // blk,),
        in_specs=[pl.BlockSpec((blk, D), lambda i: (i, 0)),
                  pl.BlockSpec((NC, blk, 1), lambda i: (0, i, 0)),
                  pl.BlockSpec((1, D), lambda i: (0, 0)),
                  pl.BlockSpec((D, hid), lambda i: (0, 0)),
                  pl.BlockSpec((1, hid), lambda i: (0, 0)),
                  pl.BlockSpec((hid, out), lambda i: (0, 0)),
                  pl.BlockSpec((1, out), lambda i: (0, 0))],
        out_specs=pl.BlockSpec((blk, out), lambda i: (i, 0)),
        out_shape=jax.ShapeDtypeStruct((N, out), jnp.float32),
    )(sums_q, cnt.reshape(NC, N_ACC, 1), b_eff.reshape(1, D),
      wc1, bc1.reshape(1, hid), wc2, bc2.reshape(1, out))


_MESH = plsc.VectorSubcoreMesh(core_axis_name="c", subcore_axis_name="s",
                               num_cores=NC, num_subcores=NS)


NBUF = 2
IH = 40
G_STEPS = IH // NBUF


def _agg_pipeline(h_hbm, acc, cacc, idxs, idxd, rows, ones, gsems, ssems,
                  csems, with_count):
    def start_gather(j, b):
        pltpu.async_copy(h_hbm.at[idxs.at[j]], rows.at[b], gsems.at[b])

    def wait_gather(b):
        pltpu.make_async_copy(h_hbm.at[idxs.at[0]], rows.at[b],
                              gsems.at[b]).wait()

    def start_scatter(j, b):
        pltpu.async_copy(rows.at[b], acc.at[idxd.at[j]], ssems.at[b],
                         add=True)
        if with_count:
            pltpu.async_copy(ones, cacc.at[idxd.at[j]], csems.at[b],
                             add=True)

    def wait_scatter(b):
        pltpu.make_async_copy(rows.at[b], acc.at[idxd.at[0]],
                              ssems.at[b]).wait()
        if with_count:
            pltpu.make_async_copy(ones, cacc.at[idxd.at[0]],
                                  csems.at[b]).wait()

    for b in range(NBUF):
        start_gather(b, b)

    def step(g, carry):
        for b in range(NBUF):
            wait_gather(b)
            start_scatter(g * NBUF + b, b)
        for b in range(NBUF):
            wait_scatter(b)
            start_gather((g + 1) * NBUF + b, b)
        return carry

    lax.fori_loop(0, G_STEPS - 1, step, 0)
    g = G_STEPS - 1
    for b in range(NBUF):
        wait_gather(b)
        start_scatter(g * NBUF + b, b)
    for b in range(NBUF):
        wait_scatter(b)


def _sc_agg_cnt_body(h_hbm, src_hbm, dst_hbm, z2_hbm, z1_hbm,
                     sums_out, cnt_out0, cnt_out1, acc, cacc, idxs, idxd,
                     rows, ones, gsems, ssems, csems):
    cid = lax.axis_index("c")
    sid = lax.axis_index("s")
    wid = sid * NC + cid
    zbase = sid * ZR
    pltpu.sync_copy(z2_hbm, acc.at[pl.ds(zbase, ZR)])
    pltpu.sync_copy(z1_hbm, cacc.at[pl.ds(zbase, ZR)])
    for i in range(CHUNK // 16):
        ones[pl.ds(i * 16, 16)] = jnp.ones((16,), jnp.float32)
    plsc.subcore_barrier()
    for half in range(CH // IH):
        pltpu.sync_copy(src_hbm.at[wid, pl.ds(half * IH, IH)], idxs)
        pltpu.sync_copy(dst_hbm.at[wid, pl.ds(half * IH, IH)], idxd)
        _agg_pipeline(h_hbm, acc, cacc, idxs, idxd, rows, ones, gsems,
                      ssems, csems, with_count=True)
    plsc.subcore_barrier()
    pltpu.sync_copy(acc.at[pl.ds(zbase, ZR)],
                    sums_out.at[cid, pl.ds(zbase, ZR)])

    @pl.when(cid == 0)
    def _():
        pltpu.sync_copy(cacc.at[pl.ds(zbase, ZR)],
                        cnt_out0.at[pl.ds(zbase, ZR)])

    @pl.when(cid == 1)
    def _():
        pltpu.sync_copy(cacc.at[pl.ds(zbase, ZR)],
                        cnt_out1.at[pl.ds(zbase, ZR)])


def _sc_agg_body(h_hbm, src_hbm, dst_hbm, z2_hbm,
                 sums_out, acc, idxs, idxd, rows, gsems, ssems):
    cid = lax.axis_index("c")
    sid = lax.axis_index("s")
    wid = sid * NC + cid
    zbase = sid * ZR
    pltpu.sync_copy(z2_hbm, acc.at[pl.ds(zbase, ZR)])
    plsc.subcore_barrier()
    for half in range(CH // IH):
        pltpu.sync_copy(src_hbm.at[wid, pl.ds(half * IH, IH)], idxs)
        pltpu.sync_copy(dst_hbm.at[wid, pl.ds(half * IH, IH)], idxd)
        _agg_pipeline(h_hbm, acc, None, idxs, idxd, rows, None, gsems,
                      ssems, None, with_count=False)
    plsc.subcore_barrier()
    pltpu.sync_copy(acc.at[pl.ds(zbase, ZR)],
                    sums_out.at[cid, pl.ds(zbase, ZR)])


_SC_AGG_CNT = pl.kernel(
    _sc_agg_cnt_body,
    out_type=(jax.ShapeDtypeStruct((NC, N_ACC, D), jnp.float32),
              jax.ShapeDtypeStruct((N_ACC,), jnp.float32),
              jax.ShapeDtypeStruct((N_ACC,), jnp.float32)),
    mesh=_MESH,
    compiler_params=pltpu.CompilerParams(use_tc_tiling_on_sc=True),
    scratch_types=[
        pltpu.VMEM_SHARED((N_ACC, D), jnp.float32),
        pltpu.VMEM_SHARED((N_ACC,), jnp.float32),
        pltpu.VMEM((IH, CHUNK), jnp.int32),
        pltpu.VMEM((IH, CHUNK), jnp.int32),
        pltpu.VMEM((NBUF, CHUNK, D), jnp.float32),
        pltpu.VMEM((CHUNK,), jnp.float32),
        pltpu.SemaphoreType.DMA((NBUF,)),
        pltpu.SemaphoreType.DMA((NBUF,)),
        pltpu.SemaphoreType.DMA((NBUF,)),
    ],
)

def _sc_agg64_body(q_hbm, src_hbm, dst_hbm, z64_hbm,
                   sums_out, acc, idxs, idxd, rows, gsems, ssems):
    cid = lax.axis_index("c")
    sid = lax.axis_index("s")
    wid = sid * NC + cid
    zbase = sid * ZR
    pltpu.sync_copy(z64_hbm, acc.at[pl.ds(zbase, ZR)])
    plsc.subcore_barrier()
    for half in range(CH // IH):
        pltpu.sync_copy(src_hbm.at[wid, pl.ds(half * IH, IH)], idxs)
        pltpu.sync_copy(dst_hbm.at[wid, pl.ds(half * IH, IH)], idxd)
        _agg_pipeline(q_hbm, acc, None, idxs, idxd, rows, None, gsems,
                      ssems, None, with_count=False)
    plsc.subcore_barrier()

    @pl.when(cid == 0)
    def _():
        pltpu.sync_copy(acc.at[pl.ds(zbase, ZR)],
                        sums_out.at[pl.ds(zbase, ZR), pl.ds(0, 64)])

    @pl.when(cid == 1)
    def _():
        pltpu.sync_copy(acc.at[pl.ds(zbase, ZR)],
                        sums_out.at[pl.ds(zbase, ZR), pl.ds(64, 64)])


_SC_AGG64 = pl.kernel(
    _sc_agg64_body,
    out_type=jax.ShapeDtypeStruct((N_ACC, D), jnp.float32),
    mesh=_MESH,
    compiler_params=pltpu.CompilerParams(use_tc_tiling_on_sc=False),
    scratch_types=[
        pltpu.VMEM_SHARED((N_ACC, 64), jnp.float32),
        pltpu.VMEM((IH, CHUNK), jnp.int32),
        pltpu.VMEM((IH, CHUNK), jnp.int32),
        pltpu.VMEM((NBUF, CHUNK, 64), jnp.float32),
        pltpu.SemaphoreType.DMA((NBUF,)),
        pltpu.SemaphoreType.DMA((NBUF,)),
    ],
)


def kernel(x, edge_index, W_feat, b_feat, group_encodings, W_agg1, b_agg1,
           W_agg2, b_agg2, W_c1, b_c1, W_c2, b_c2):
    src = edge_index[0]
    dst = edge_index[1]
    e = src.shape[0]
    pad = E_PAD - e
    ar = lax.iota(jnp.int32, pad)
    src3d = jnp.concatenate([src, ar % N]).reshape(NW, CH, CHUNK)
    dst3d = jnp.concatenate(
        [dst, N + ar % (N_ACC - N)]).reshape(NW, CH, CHUNK)
    z2 = jnp.zeros((ZR, D), jnp.float32)
    z1 = jnp.zeros((ZR,), jnp.float32)
    z64 = jnp.zeros((ZR, 64), jnp.float32)
    b_eff = b_agg2 + jnp.mean(group_encodings, axis=0)

    h = _tc_linear_relu(x, W_feat, b_feat)
    sums1, cnt0, cnt1 = _SC_AGG_CNT(h, src3d, dst3d, z2, z1)
    cnt = jnp.stack([cnt0, cnt1], axis=0)
    q = _tc_norm_lin_q(sums1, cnt, W_agg1, b_agg1, W_agg2, W_c1)
    sums_q = _SC_AGG64(q, src3d, dst3d, z64)
    return _tc_final(sums_q, cnt, b_eff, W_c1, b_c1, W_c2, b_c2)


# speedup vs baseline: 13.4302x; 1.0810x over previous
import jax
import jax.numpy as jnp
from jax import lax
from jax.experimental import pallas as pl
from jax.experimental.pallas import tpu as pltpu
from jax.experimental.pallas import tpu_sc as plsc

N = 10000
D = 128
NC = 2
NS = 16
NW = NC * NS
CHUNK = 128
CH = 80
E_PAD = NW * CH * CHUNK
N_ACC = 10240
ZR = N_ACC // NS
ROWBLK = 1280


def _lin_relu_body(x_ref, w_ref, b_ref, o_ref):
    o_ref[...] = jnp.maximum(
        jnp.dot(x_ref[...], w_ref[...], preferred_element_type=jnp.float32)
        + b_ref[...], 0.0)


def _tc_linear_relu(x, w, b):
    n = x.shape[0]
    blk = 1000
    return pl.pallas_call(
        _lin_relu_body,
        grid=(n // blk,),
        in_specs=[pl.BlockSpec((blk, D), lambda i: (i, 0)),
                  pl.BlockSpec((D, D), lambda i: (0, 0)),
                  pl.BlockSpec((1, D), lambda i: (0, 0))],
        out_specs=pl.BlockSpec((blk, D), lambda i: (i, 0)),
        out_shape=jax.ShapeDtypeStruct((n, D), jnp.float32),
    )(x, w, b.reshape(1, D))


def _norm_lin_relu_body(s_ref, c_ref, w_ref, b_ref, o_ref):
    cnt = c_ref[0] + c_ref[1]
    m = (s_ref[0] + s_ref[1]) / jnp.maximum(cnt, 1.0)
    o_ref[...] = jnp.maximum(
        jnp.dot(m, w_ref[...], preferred_element_type=jnp.float32)
        + b_ref[...], 0.0)


def _tc_norm_linear_relu(sums, cnt, w, b):
    return pl.pallas_call(
        _norm_lin_relu_body,
        grid=(N_ACC // ROWBLK,),
        in_specs=[pl.BlockSpec((NC, ROWBLK, D), lambda i: (0, i, 0)),
                  pl.BlockSpec((NC, ROWBLK, 1), lambda i: (0, i, 0)),
                  pl.BlockSpec((D, D), lambda i: (0, 0)),
                  pl.BlockSpec((1, D), lambda i: (0, 0))],
        out_specs=pl.BlockSpec((ROWBLK, D), lambda i: (i, 0)),
        out_shape=jax.ShapeDtypeStruct((N_ACC, D), jnp.float32),
    )(sums, cnt.reshape(NC, N_ACC, 1), w, b.reshape(1, D))


def _norm_lin_q_body(s_ref, c_ref, w1_ref, b1_ref, w2_ref, wc1_ref, o_ref):
    cnt = c_ref[0] + c_ref[1]
    m = (s_ref[0] + s_ref[1]) / jnp.maximum(cnt, 1.0)
    h1 = jnp.maximum(
        jnp.dot(m, w1_ref[...], preferred_element_type=jnp.float32)
        + b1_ref[...], 0.0)
    wq = jnp.dot(w2_ref[...], wc1_ref[...],
                 preferred_element_type=jnp.float32,
                 precision=lax.Precision.HIGHEST)
    o_ref[...] = jnp.dot(h1, wq, preferred_element_type=jnp.float32,
                         precision=lax.Precision.HIGHEST)


def _tc_norm_lin_q(sums, cnt, w1, b1, w2, wc1):
    hid = wc1.shape[1]
    return pl.pallas_call(
        _norm_lin_q_body,
        grid=(N_ACC // ROWBLK,),
        in_specs=[pl.BlockSpec((NC, ROWBLK, D), lambda i: (0, i, 0)),
                  pl.BlockSpec((NC, ROWBLK, 1), lambda i: (0, i, 0)),
                  pl.BlockSpec((D, D), lambda i: (0, 0)),
                  pl.BlockSpec((1, D), lambda i: (0, 0)),
                  pl.BlockSpec((D, D), lambda i: (0, 0)),
                  pl.BlockSpec((D, hid), lambda i: (0, 0))],
        out_specs=pl.BlockSpec((ROWBLK, hid), lambda i: (i, 0)),
        out_shape=jax.ShapeDtypeStruct((N_ACC, hid), jnp.float32),
    )(sums, cnt.reshape(NC, N_ACC, 1), w1, b1.reshape(1, D), w2, wc1)


def _final_body(s_ref, c_ref, beff_ref, wc1_ref, bc1_ref, wc2_ref,
                bc2_ref, o_ref):
    cnt = c_ref[0] + c_ref[1]
    m = (s_ref[:, 0:64] + s_ref[:, 64:128]) / jnp.maximum(cnt, 1.0)
    cb = jnp.dot(beff_ref[...], wc1_ref[...],
                 preferred_element_type=jnp.float32,
                 precision=lax.Precision.HIGHEST) + bc1_ref[...]
    hid = jnp.maximum(m + cb, 0.0)
    o_ref[...] = jnp.dot(hid, wc2_ref[...],
                         preferred_element_type=jnp.float32) + bc2_ref[...]


def _tc_final(sums_q, cnt, b_eff, wc1, bc1, wc2, bc2):
    hid = wc1.shape[1]
    out = wc2.shape[1]
    blk = 1000
    return pl.pallas_call(
        _final_body,
        grid=(N // blk,),
        in_specs=[pl.BlockSpec((blk, D), lambda i: (i, 0)),
                  pl.BlockSpec((NC, blk, 1), lambda i: (0, i, 0)),
                  pl.BlockSpec((1, D), lambda i: (0, 0)),
                  pl.BlockSpec((D, hid), lambda i: (0, 0)),
                  pl.BlockSpec((1, hid), lambda i: (0, 0)),
                  pl.BlockSpec((hid, out), lambda i: (0, 0)),
                  pl.BlockSpec((1, out), lambda i: (0, 0))],
        out_specs=pl.BlockSpec((blk, out), lambda i: (i, 0)),
        out_shape=jax.ShapeDtypeStruct((N, out), jnp.float32),
    )(sums_q, cnt.reshape(NC, N_ACC, 1), b_eff.reshape(1, D),
      wc1, bc1.reshape(1, hid), wc2, bc2.reshape(1, out))


_MESH = plsc.VectorSubcoreMesh(core_axis_name="c", subcore_axis_name="s",
                               num_cores=NC, num_subcores=NS)


NBUF = 2
NBUF64 = 4
IH = 40


def _agg_pipeline(h_hbm, acc, cacc, idxs, idxd, rows, ones, gsems, ssems,
                  csems, with_count, nbuf=NBUF):
    g_steps = IH // nbuf

    def start_gather(j, b):
        pltpu.async_copy(h_hbm.at[idxs.at[j]], rows.at[b], gsems.at[b])

    def wait_gather(b):
        pltpu.make_async_copy(h_hbm.at[idxs.at[0]], rows.at[b],
                              gsems.at[b]).wait()

    def start_scatter(j, b):
        pltpu.async_copy(rows.at[b], acc.at[idxd.at[j]], ssems.at[b],
                         add=True)
        if with_count:
            pltpu.async_copy(ones, cacc.at[idxd.at[j]], csems.at[b],
                             add=True)

    def wait_scatter(b):
        pltpu.make_async_copy(rows.at[b], acc.at[idxd.at[0]],
                              ssems.at[b]).wait()
        if with_count:
            pltpu.make_async_copy(ones, cacc.at[idxd.at[0]],
                                  csems.at[b]).wait()

    for b in range(nbuf):
        start_gather(b, b)

    def step(g, carry):
        for b in range(nbuf):
            wait_gather(b)
            start_scatter(g * nbuf + b, b)
        for b in range(nbuf):
            wait_scatter(b)
            start_gather((g + 1) * nbuf + b, b)
        return carry

    lax.fori_loop(0, g_steps - 1, step, 0)
    g = g_steps - 1
    for b in range(nbuf):
        wait_gather(b)
        start_scatter(g * nbuf + b, b)
    for b in range(nbuf):
        wait_scatter(b)


def _sc_agg_cnt_body(h_hbm, src_hbm, dst_hbm, z2_hbm, z1_hbm,
                     sums_out, cnt_out0, cnt_out1, acc, cacc, idxs, idxd,
                     rows, ones, gsems, ssems, csems):
    cid = lax.axis_index("c")
    sid = lax.axis_index("s")
    wid = sid * NC + cid
    zbase = sid * ZR
    pltpu.sync_copy(z2_hbm, acc.at[pl.ds(zbase, ZR)])
    pltpu.sync_copy(z1_hbm, cacc.at[pl.ds(zbase, ZR)])
    for i in range(CHUNK // 16):
        ones[pl.ds(i * 16, 16)] = jnp.ones((16,), jnp.float32)
    plsc.subcore_barrier()
    for half in range(CH // IH):
        pltpu.sync_copy(src_hbm.at[wid, pl.ds(half * IH, IH)], idxs)
        pltpu.sync_copy(dst_hbm.at[wid, pl.ds(half * IH, IH)], idxd)
        _agg_pipeline(h_hbm, acc, cacc, idxs, idxd, rows, ones, gsems,
                      ssems, csems, with_count=True)
    plsc.subcore_barrier()
    pltpu.sync_copy(acc.at[pl.ds(zbase, ZR)],
                    sums_out.at[cid, pl.ds(zbase, ZR)])

    @pl.when(cid == 0)
    def _():
        pltpu.sync_copy(cacc.at[pl.ds(zbase, ZR)],
                        cnt_out0.at[pl.ds(zbase, ZR)])

    @pl.when(cid == 1)
    def _():
        pltpu.sync_copy(cacc.at[pl.ds(zbase, ZR)],
                        cnt_out1.at[pl.ds(zbase, ZR)])


def _sc_agg_body(h_hbm, src_hbm, dst_hbm, z2_hbm,
                 sums_out, acc, idxs, idxd, rows, gsems, ssems):
    cid = lax.axis_index("c")
    sid = lax.axis_index("s")
    wid = sid * NC + cid
    zbase = sid * ZR
    pltpu.sync_copy(z2_hbm, acc.at[pl.ds(zbase, ZR)])
    plsc.subcore_barrier()
    for half in range(CH // IH):
        pltpu.sync_copy(src_hbm.at[wid, pl.ds(half * IH, IH)], idxs)
        pltpu.sync_copy(dst_hbm.at[wid, pl.ds(half * IH, IH)], idxd)
        _agg_pipeline(h_hbm, acc, None, idxs, idxd, rows, None, gsems,
                      ssems, None, with_count=False)
    plsc.subcore_barrier()
    pltpu.sync_copy(acc.at[pl.ds(zbase, ZR)],
                    sums_out.at[cid, pl.ds(zbase, ZR)])


_SC_AGG_CNT = pl.kernel(
    _sc_agg_cnt_body,
    out_type=(jax.ShapeDtypeStruct((NC, N_ACC, D), jnp.float32),
              jax.ShapeDtypeStruct((N_ACC,), jnp.float32),
              jax.ShapeDtypeStruct((N_ACC,), jnp.float32)),
    mesh=_MESH,
    compiler_params=pltpu.CompilerParams(use_tc_tiling_on_sc=True),
    scratch_types=[
        pltpu.VMEM_SHARED((N_ACC, D), jnp.float32),
        pltpu.VMEM_SHARED((N_ACC,), jnp.float32),
        pltpu.VMEM((IH, CHUNK), jnp.int32),
        pltpu.VMEM((IH, CHUNK), jnp.int32),
        pltpu.VMEM((NBUF, CHUNK, D), jnp.float32),
        pltpu.VMEM((CHUNK,), jnp.float32),
        pltpu.SemaphoreType.DMA((NBUF,)),
        pltpu.SemaphoreType.DMA((NBUF,)),
        pltpu.SemaphoreType.DMA((NBUF,)),
    ],
)

def _sc_agg64_body(q_hbm, src_hbm, dst_hbm, z64_hbm,
                   sums_out, acc, idxs, idxd, rows, gsems, ssems):
    cid = lax.axis_index("c")
    sid = lax.axis_index("s")
    wid = sid * NC + cid
    zbase = sid * ZR
    pltpu.sync_copy(z64_hbm, acc.at[pl.ds(zbase, ZR)])
    plsc.subcore_barrier()
    for half in range(CH // IH):
        pltpu.sync_copy(src_hbm.at[wid, pl.ds(half * IH, IH)], idxs)
        pltpu.sync_copy(dst_hbm.at[wid, pl.ds(half * IH, IH)], idxd)
        _agg_pipeline(q_hbm, acc, None, idxs, idxd, rows, None, gsems,
                      ssems, None, with_count=False, nbuf=NBUF64)
    plsc.subcore_barrier()

    @pl.when(cid == 0)
    def _():
        pltpu.sync_copy(acc.at[pl.ds(zbase, ZR)],
                        sums_out.at[pl.ds(zbase, ZR), pl.ds(0, 64)])

    @pl.when(cid == 1)
    def _():
        pltpu.sync_copy(acc.at[pl.ds(zbase, ZR)],
                        sums_out.at[pl.ds(zbase, ZR), pl.ds(64, 64)])


_SC_AGG64 = pl.kernel(
    _sc_agg64_body,
    out_type=jax.ShapeDtypeStruct((N_ACC, D), jnp.float32),
    mesh=_MESH,
    compiler_params=pltpu.CompilerParams(use_tc_tiling_on_sc=False),
    scratch_types=[
        pltpu.VMEM_SHARED((N_ACC, 64), jnp.float32),
        pltpu.VMEM((IH, CHUNK), jnp.int32),
        pltpu.VMEM((IH, CHUNK), jnp.int32),
        pltpu.VMEM((NBUF64, CHUNK, 64), jnp.float32),
        pltpu.SemaphoreType.DMA((NBUF64,)),
        pltpu.SemaphoreType.DMA((NBUF64,)),
    ],
)


def kernel(x, edge_index, W_feat, b_feat, group_encodings, W_agg1, b_agg1,
           W_agg2, b_agg2, W_c1, b_c1, W_c2, b_c2):
    src = edge_index[0]
    dst = edge_index[1]
    e = src.shape[0]
    pad = E_PAD - e
    ar = lax.iota(jnp.int32, pad)
    src3d = jnp.concatenate([src, ar % N]).reshape(NW, CH, CHUNK)
    dst3d = jnp.concatenate(
        [dst, N + ar % (N_ACC - N)]).reshape(NW, CH, CHUNK)
    z2 = jnp.zeros((ZR, D), jnp.float32)
    z1 = jnp.zeros((ZR,), jnp.float32)
    z64 = jnp.zeros((ZR, 64), jnp.float32)
    b_eff = b_agg2 + jnp.mean(group_encodings, axis=0)

    h = _tc_linear_relu(x, W_feat, b_feat)
    sums1, cnt0, cnt1 = _SC_AGG_CNT(h, src3d, dst3d, z2, z1)
    cnt = jnp.stack([cnt0, cnt1], axis=0)
    q = _tc_norm_lin_q(sums1, cnt, W_agg1, b_agg1, W_agg2, W_c1)
    sums_q = _SC_AGG64(q, src3d, dst3d, z64)
    return _tc_final(sums_q, cnt, b_eff, W_c1, b_c1, W_c2, b_c2)
